# Initial kernel scaffold; baseline (speedup 1.0000x reference)
import jax
import jax.numpy as jnp
from jax.experimental import pallas as pl


def kernel(h, e, edge_index, node2graph, params):
    raise NotImplementedError("write your pallas kernel here")


# speedup vs baseline: 7.4019x; 7.4019x over previous
import functools

import jax
import jax.numpy as jnp
from jax import lax
from jax.experimental import pallas as pl
from jax.experimental.pallas import tpu as pltpu
from jax.experimental.pallas import tpu_sc as plsc

N = 10000
E = 320000
G = 256
H = 128
BN = 400
NBN = N // BN
BE = 512
NBE = E // BE
NW = 32
EPW = E // NW
CH = 80
NCH = EPW // CH
NPAD = 10240
RPS = NPAD // 16
ZCH = 128

_f32 = jnp.float32
_i32 = jnp.int32


def _lrelu(x):
    return jnp.where(x >= 0, x, 0.01 * x)


def _elu(x):
    return jnp.where(x > 0, x, jnp.exp(jnp.minimum(x, 0.0)) - 1.0)


def _pack8(*rows):
    rows = list(rows) + [jnp.zeros((128,), _f32)] * (8 - len(rows))
    return jnp.stack(rows)


def _pack8w(*rows):
    rows = list(rows) + [jnp.zeros((384,), _f32)] * (8 - len(rows))
    return jnp.stack(rows)


def _gru(x, hs, wihT, whhT, bih, bhh):
    gi = lax.dot_general(x, wihT, (((1,), (0,)), ((), ())),
                         preferred_element_type=_f32) + bih
    gh = lax.dot_general(hs, whhT, (((1,), (0,)), ((), ())),
                         preferred_element_type=_f32) + bhh
    r = jax.nn.sigmoid(gi[:, 0:128] + gh[:, 0:128])
    z = jax.nn.sigmoid(gi[:, 128:256] + gh[:, 128:256])
    n = jnp.tanh(gi[:, 256:384] + r * gh[:, 256:384])
    return (1.0 - z) * n + z * hs


def _dot(a, b):
    return lax.dot_general(a, b, (((1,), (0,)), ((), ())),
                           preferred_element_type=_f32)


def _t1_body(h_r, wpn_r, bpn_r, wpe1n_r, scal_r, hv_r, hn_r, sd_r):
    x = h_r[...]
    hv = _lrelu(_dot(x, wpn_r[...]) + bpn_r[0:1, :])
    hv_r[...] = hv
    hn_r[...] = _dot(x, wpe1n_r[...])
    sd = jnp.sum(hv * scal_r[0:1, :], axis=1, keepdims=True) + scal_r[1:2, 0:1]
    sd_r[...] = jnp.broadcast_to(sd, (BN, H))


def _t1(h_pad, wpn, bpn, wpe1n, scal):
    return pl.pallas_call(
        _t1_body,
        grid=(NBN,),
        in_specs=[
            pl.BlockSpec((BN, 256), lambda i: (i, 0)),
            pl.BlockSpec((256, H), lambda i: (0, 0)),
            pl.BlockSpec((8, H), lambda i: (0, 0)),
            pl.BlockSpec((256, H), lambda i: (0, 0)),
            pl.BlockSpec((8, H), lambda i: (0, 0)),
        ],
        out_specs=[
            pl.BlockSpec((BN, H), lambda i: (i, 0)),
            pl.BlockSpec((BN, H), lambda i: (i, 0)),
            pl.BlockSpec((BN, H), lambda i: (i, 0)),
        ],
        out_shape=[jax.ShapeDtypeStruct((N, H), _f32)] * 3,
    )(h_pad, wpn, bpn, wpe1n, scal)


def _t2_body(e_r, we_r, be_r, ee_r):
    ee_r[...] = lax.dot_general(e_r[...], we_r[...], (((1,), (1,)), ((), ())),
                                preferred_element_type=_f32) + be_r[0:1, :]


def _t2(e, we, be):
    fe = e.shape[1]
    return pl.pallas_call(
        _t2_body,
        grid=(NBE,),
        in_specs=[
            pl.BlockSpec((BE, fe), lambda i: (i, 0)),
            pl.BlockSpec((H, fe), lambda i: (0, 0)),
            pl.BlockSpec((8, H), lambda i: (0, 0)),
        ],
        out_specs=pl.BlockSpec((BE, H), lambda i: (i, 0)),
        out_shape=jax.ShapeDtypeStruct((E, H), _f32),
    )(e, we, be)


def _t3_body(he_r, wet_r, bet_r, w2b_r, m_r, se_r, mse_r, mx_s):
    i = pl.program_id(0)
    h1 = he_r[...]
    m_r[...] = _dot(h1, wet_r[...]) + bet_r[0:1, :]
    se_row = lax.dot_general(w2b_r[0:1, :], h1, (((1,), (1,)), ((), ())),
                             preferred_element_type=_f32)
    se_r[...] = se_row.reshape(1, 1, BE)
    bmax = jnp.max(se_row)

    @pl.when(i == 0)
    def _():
        mx_s[0] = bmax

    @pl.when(i > 0)
    def _():
        mx_s[0] = jnp.maximum(mx_s[0], bmax)

    @pl.when(i == NBE - 1)
    def _():
        mse_r[...] = jnp.broadcast_to(jnp.maximum(mx_s[0], bmax), (8, H))


def _t3(he1, wet, bet, w2b):
    return pl.pallas_call(
        _t3_body,
        grid=(NBE,),
        in_specs=[
            pl.BlockSpec((BE, H), lambda i: (i, 0)),
            pl.BlockSpec((H, H), lambda i: (0, 0)),
            pl.BlockSpec((8, H), lambda i: (0, 0)),
            pl.BlockSpec((8, H), lambda i: (0, 0)),
        ],
        out_specs=[
            pl.BlockSpec((BE, H), lambda i: (i, 0)),
            pl.BlockSpec((1, 1, BE), lambda i: (i, 0, 0)),
            pl.BlockSpec((8, H), lambda i: (0, 0)),
        ],
        out_shape=[
            jax.ShapeDtypeStruct((E, H), _f32),
            jax.ShapeDtypeStruct((NBE, 1, BE), _f32),
            jax.ShapeDtypeStruct((8, H), _f32),
        ],
        scratch_shapes=[pltpu.SMEM((1,), _f32)],
    )(he1, wet, bet, w2b)


def _t45_body(tbl_r, den_r, hs_r, wihT_r, whhT_r, bih_r, bhh_r, scal_r,
              wpnT_r, bpn_r, nf_r, sd_r, ss_r, hvp_r, mss_r, mx_s):
    i = pl.program_id(0)
    t = tbl_r[...]
    numer = t[0] + t[1]
    d = den_r[...]
    den = d[0] + d[1]
    ctx = _elu(numer / (den + 1e-9))
    nf = jax.nn.relu(_gru(ctx, hs_r[...], wihT_r[...], whhT_r[...],
                          bih_r[0:1, :], bhh_r[0:1, :]))
    nf_r[...] = nf
    sd = jnp.sum(nf * scal_r[0:1, :], axis=1, keepdims=True) + scal_r[2:3, 0:1]
    sd_r[...] = jnp.broadcast_to(sd, (BN, H))
    ss = jnp.sum(nf * scal_r[1:2, :], axis=1, keepdims=True)
    ss_r[...] = jnp.broadcast_to(ss, (BN, H))
    hvp_r[...] = _dot(nf, wpnT_r[...]) + bpn_r[0:1, :]
    bmax = jnp.max(ss)

    @pl.when(i == 0)
    def _():
        mx_s[0] = bmax

    @pl.when(i > 0)
    def _():
        mx_s[0] = jnp.maximum(mx_s[0], bmax)

    @pl.when(i == NBN - 1)
    def _():
        mss_r[...] = jnp.broadcast_to(jnp.maximum(mx_s[0], bmax), (8, H))


def _t45(tbl, den, hs, wihT, whhT, bih, bhh, scal, wpnT, bpn):
    return pl.pallas_call(
        _t45_body,
        grid=(NBN,),
        in_specs=[
            pl.BlockSpec((2, BN, H), lambda i: (0, i, 0)),
            pl.BlockSpec((2, BN, 1), lambda i: (0, i, 0)),
            pl.BlockSpec((BN, H), lambda i: (i, 0)),
            pl.BlockSpec((H, 384), lambda i: (0, 0)),
            pl.BlockSpec((H, 384), lambda i: (0, 0)),
            pl.BlockSpec((8, 384), lambda i: (0, 0)),
            pl.BlockSpec((8, 384), lambda i: (0, 0)),
            pl.BlockSpec((8, H), lambda i: (0, 0)),
            pl.BlockSpec((H, H), lambda i: (0, 0)),
            pl.BlockSpec((8, H), lambda i: (0, 0)),
        ],
        out_specs=[
            pl.BlockSpec((BN, H), lambda i: (i, 0)),
            pl.BlockSpec((BN, H), lambda i: (i, 0)),
            pl.BlockSpec((BN, H), lambda i: (i, 0)),
            pl.BlockSpec((BN, H), lambda i: (i, 0)),
            pl.BlockSpec((8, H), lambda i: (0, 0)),
        ],
        out_shape=[
            jax.ShapeDtypeStruct((N, H), _f32),
            jax.ShapeDtypeStruct((N, H), _f32),
            jax.ShapeDtypeStruct((N, H), _f32),
            jax.ShapeDtypeStruct((N, H), _f32),
            jax.ShapeDtypeStruct((8, H), _f32),
        ],
        scratch_shapes=[pltpu.SMEM((1,), _f32)],
    )(tbl, den, hs, wihT, whhT, bih, bhh, scal, wpnT, bpn)


def _t6_body(tbl_r, den_r, hs_r, wihT_r, whhT_r, bih_r, bhh_r, n2g_r, clws_r,
             h_r, g0_r, mhs_r, mx_s, acc_v):
    i = pl.program_id(0)
    t = tbl_r[...]
    numer = t[0] + t[1]
    d = den_r[...]
    den = d[0] + d[1]
    ctx = _elu(numer / (den + 1e-9))
    nf = jax.nn.relu(_gru(ctx, hs_r[...], wihT_r[...], whhT_r[...],
                          bih_r[0:1, :], bhh_r[0:1, :]))
    h_r[...] = nf
    n2g = n2g_r[0, 0, :]
    oh = (n2g[:, None] == lax.broadcasted_iota(_i32, (BN, G), 1)).astype(_f32)

    @pl.when(i == 0)
    def _():
        acc_v[...] = jnp.zeros((G, H), _f32)

    acc_v[...] += lax.dot_general(oh, nf, (((0,), (0,)), ((), ())),
                                  preferred_element_type=_f32)
    hs0 = jnp.max(jnp.sum(nf * clws_r[0:1, :], axis=1))
    hs1 = jnp.max(jnp.sum(nf * clws_r[1:2, :], axis=1))

    @pl.when(i == 0)
    def _():
        mx_s[0] = hs0
        mx_s[1] = hs1

    @pl.when(i > 0)
    def _():
        mx_s[0] = jnp.maximum(mx_s[0], hs0)
        mx_s[1] = jnp.maximum(mx_s[1], hs1)

    @pl.when(i == NBN - 1)
    def _():
        g0_r[...] = acc_v[...]
        m0 = jnp.maximum(mx_s[0], hs0)
        m1 = jnp.maximum(mx_s[1], hs1)
        mhs_r[...] = jnp.broadcast_to(
            jnp.stack([m0, m1] + [jnp.float32(0.0)] * 6)[:, None], (8, H))


def _t6(tbl, den, hs, wihT, whhT, bih, bhh, n2g3, clws):
    return pl.pallas_call(
        _t6_body,
        grid=(NBN,),
        in_specs=[
            pl.BlockSpec((2, BN, H), lambda i: (0, i, 0)),
            pl.BlockSpec((2, BN, 1), lambda i: (0, i, 0)),
            pl.BlockSpec((BN, H), lambda i: (i, 0)),
            pl.BlockSpec((H, 384), lambda i: (0, 0)),
            pl.BlockSpec((H, 384), lambda i: (0, 0)),
            pl.BlockSpec((8, 384), lambda i: (0, 0)),
            pl.BlockSpec((8, 384), lambda i: (0, 0)),
            pl.BlockSpec((1, 1, BN), lambda i: (i, 0, 0)),
            pl.BlockSpec((8, H), lambda i: (0, 0)),
        ],
        out_specs=[
            pl.BlockSpec((BN, H), lambda i: (i, 0)),
            pl.BlockSpec((G, H), lambda i: (0, 0)),
            pl.BlockSpec((8, H), lambda i: (0, 0)),
        ],
        out_shape=[
            jax.ShapeDtypeStruct((N, H), _f32),
            jax.ShapeDtypeStruct((G, H), _f32),
            jax.ShapeDtypeStruct((8, H), _f32),
        ],
        scratch_shapes=[pltpu.SMEM((2,), _f32), pltpu.VMEM((G, H), _f32)],
    )(tbl, den, hs, wihT, whhT, bih, bhh, n2g3, clws)


def _t78_body(h_r, n2g_r, gf_r, scal_r, wpnT_r, bpn_r, wihT_r, whhT_r,
              bih_r, bhh_r, out_r, gsb_v, nacc_v, dacc_v):
    i = pl.program_id(0)

    @pl.when(i == 0)
    def _():
        gf = gf_r[...]
        gs = jnp.sum(jax.nn.relu(gf) * scal_r[0:1, :], axis=1,
                     keepdims=True) + scal_r[2:3, 0:1]
        gsb_v[...] = jnp.broadcast_to(gs, (G, H))
        nacc_v[...] = jnp.zeros((G, H), _f32)
        dacc_v[...] = jnp.zeros((G, H), _f32)

    hx = h_r[...]
    n2g = n2g_r[0, 0, :]
    oh = (n2g[:, None] == lax.broadcasted_iota(_i32, (BN, G), 1)).astype(_f32)
    gath = lax.dot_general(oh, gsb_v[...], (((1,), (0,)), ((), ())),
                           preferred_element_type=_f32)[:, 0:1]
    hscal = jnp.sum(hx * scal_r[1:2, :], axis=1, keepdims=True)
    z = _lrelu(gath + hscal)
    bd = _lrelu(gath + scal_r[3:4, 0:1])
    ex = jnp.exp(z - bd)
    hvp = _dot(hx, wpnT_r[...]) + bpn_r[0:1, :]
    nacc_v[...] += lax.dot_general(oh, ex * hvp, (((0,), (0,)), ((), ())),
                                   preferred_element_type=_f32)
    dacc_v[...] += lax.dot_general(oh, jnp.broadcast_to(ex, (BN, H)),
                                   (((0,), (0,)), ((), ())),
                                   preferred_element_type=_f32)

    @pl.when(i == NBN - 1)
    def _():
        grep = _elu(nacc_v[...] / (dacc_v[...][:, 0:1] + 1e-9))
        ctx = jax.nn.relu(grep)
        out_r[...] = _gru(ctx, gf_r[...], wihT_r[...], whhT_r[...],
                          bih_r[0:1, :], bhh_r[0:1, :])


def _t78(h_out, n2g3, gf, scal, wpnT, bpn, wihT, whhT, bih, bhh):
    return pl.pallas_call(
        _t78_body,
        grid=(NBN,),
        in_specs=[
            pl.BlockSpec((BN, H), lambda i: (i, 0)),
            pl.BlockSpec((1, 1, BN), lambda i: (i, 0, 0)),
            pl.BlockSpec((G, H), lambda i: (0, 0)),
            pl.BlockSpec((8, H), lambda i: (0, 0)),
            pl.BlockSpec((H, H), lambda i: (0, 0)),
            pl.BlockSpec((8, H), lambda i: (0, 0)),
            pl.BlockSpec((H, 384), lambda i: (0, 0)),
            pl.BlockSpec((H, 384), lambda i: (0, 0)),
            pl.BlockSpec((8, 384), lambda i: (0, 0)),
            pl.BlockSpec((8, 384), lambda i: (0, 0)),
        ],
        out_specs=pl.BlockSpec((G, H), lambda i: (0, 0)),
        out_shape=jax.ShapeDtypeStruct((G, H), _f32),
        scratch_shapes=[pltpu.VMEM((G, H), _f32), pltpu.VMEM((G, H), _f32),
                        pltpu.VMEM((G, H), _f32)],
    )(h_out, n2g3, gf, scal, wpnT, bpn, wihT, whhT, bih, bhh)


_SC_MESH = plsc.VectorSubcoreMesh(core_axis_name="c", subcore_axis_name="s")


@functools.partial(
    pl.kernel,
    out_type=jax.ShapeDtypeStruct((E, H), _f32),
    mesh=_SC_MESH,
    scratch_types=[
        pltpu.VMEM((CH,), _i32),
        pltpu.VMEM((CH, H), _f32),
        pltpu.VMEM((CH, H), _f32),
        pltpu.SemaphoreType.DMA,
    ],
)
def _sc_he1(hn_h, src_h, ee_h, out_h, idx_v, ra_v, rb_v, sem):
    core = lax.axis_index("c")
    sub = lax.axis_index("s")
    wid = core * 16 + sub

    def chunk(ci, carry):
        base = wid * EPW + ci * CH
        pltpu.sync_copy(src_h.at[pl.ds(base, CH)], idx_v)
        cp = pltpu.async_copy(hn_h.at[idx_v], ra_v, sem)
        pltpu.sync_copy(ee_h.at[pl.ds(base, CH)], rb_v)
        cp.wait()

        def row(r, c2):
            for k in range(8):
                a = ra_v[r, pl.ds(k * 16, 16)]
                b = rb_v[r, pl.ds(k * 16, 16)]
                x = a + b
                ra_v[r, pl.ds(k * 16, 16)] = jnp.where(x >= 0, x, 0.01 * x)
            return c2

        lax.fori_loop(0, CH, row, 0)
        pltpu.sync_copy(ra_v, out_h.at[pl.ds(base, CH)])
        return carry

    lax.fori_loop(0, NCH, chunk, 0)


def _sc_agg_body(rows_linear, dst_h, sd_h, mv_h, sev_src_h, rows_src_h,
                 src_h, out_h, outd_h, didx_v, sidx_v, sdv_v, sev_v, exv_v,
                 rows_v, scaled_v, mv_v, zbuf_v, dbuf_v, tbl_sh, den_sh, sem):
    core = lax.axis_index("c")
    sub = lax.axis_index("s")
    wid = core * 16 + sub

    def zrow(r, c):
        for k in range(H // 16):
            zbuf_v[r, pl.ds(k * 16, 16)] = jnp.zeros((16,), _f32)
        return c

    lax.fori_loop(0, ZCH, zrow, 0)
    for i in range(5):
        pltpu.sync_copy(zbuf_v, tbl_sh.at[pl.ds(sub * RPS + i * ZCH, ZCH)])

    def zden(r, c):
        dbuf_v[pl.ds(r * 16, 16)] = jnp.zeros((16,), _f32)
        return c

    lax.fori_loop(0, RPS // 16, zden, 0)
    pltpu.sync_copy(dbuf_v, den_sh.at[pl.ds(sub * RPS, RPS)])
    plsc.subcore_barrier()

    pltpu.sync_copy(mv_h, mv_v)

    def chunk(ci, carry):
        base = wid * EPW + ci * CH
        pltpu.sync_copy(dst_h.at[pl.ds(base, CH)], didx_v)
        g1 = pltpu.async_copy(sd_h.at[didx_v], sdv_v, sem)
        if rows_linear:
            pltpu.sync_copy(sev_src_h.at[pl.ds(base, CH)], sev_v)
            pltpu.sync_copy(rows_src_h.at[pl.ds(base, CH)], rows_v)
            g1.wait()
        else:
            pltpu.sync_copy(src_h.at[pl.ds(base, CH)], sidx_v)
            g2 = pltpu.async_copy(sev_src_h.at[sidx_v], sev_v, sem)
            g3 = pltpu.async_copy(rows_src_h.at[sidx_v], rows_v, sem)
            g1.wait()
            g2.wait()
            g3.wait()
        mv = mv_v[...]
        for g in range(CH // 16):
            s_d = sdv_v[pl.ds(g * 16, 16)]
            s_e = sev_v[pl.ds(g * 16, 16)]
            x = s_d + s_e
            lr = jnp.where(x >= 0, x, 0.01 * x)
            t2 = s_d + mv
            lb = jnp.where(t2 >= 0, t2, 0.01 * t2)
            exv_v[pl.ds(g * 16, 16)] = jnp.exp(lr - lb)
        for g in range(CH // 16):
            exg = exv_v[pl.ds(g * 16, 16)]
            for j in range(16):
                ei = g * 16 + j
                b = jnp.full((16,), exg[j], _f32)
                for k in range(H // 16):
                    scaled_v[ei, pl.ds(k * 16, 16)] = (
                        rows_v[ei, pl.ds(k * 16, 16)] * b)
        pltpu.sync_copy(scaled_v, tbl_sh.at[didx_v], add=True)
        pltpu.sync_copy(exv_v, den_sh.at[didx_v], add=True)
        return carry

    lax.fori_loop(0, NCH, chunk, 0)
    plsc.subcore_barrier()
    for i in range(5):
        pltpu.sync_copy(tbl_sh.at[pl.ds(sub * RPS + i * ZCH, ZCH)], zbuf_v)
        pltpu.sync_copy(zbuf_v, out_h.at[core, pl.ds(sub * RPS + i * ZCH, ZCH)])
    pltpu.sync_copy(den_sh.at[pl.ds(sub * RPS, RPS)], dbuf_v)
    pltpu.sync_copy(dbuf_v, outd_h.at[core, pl.ds(sub * RPS, RPS)])


def _make_sc_agg(rows_linear):
    return functools.partial(
        pl.kernel,
        out_type=[jax.ShapeDtypeStruct((2, NPAD, H), _f32),
                  jax.ShapeDtypeStruct((2, NPAD), _f32)],
        mesh=_SC_MESH,
        scratch_types=[
            pltpu.VMEM((CH,), _i32),
            pltpu.VMEM((CH,), _i32),
            pltpu.VMEM((CH,), _f32),
            pltpu.VMEM((CH,), _f32),
            pltpu.VMEM((CH,), _f32),
            pltpu.VMEM((CH, H), _f32),
            pltpu.VMEM((CH, H), _f32),
            pltpu.VMEM((16,), _f32),
            pltpu.VMEM((ZCH, H), _f32),
            pltpu.VMEM((RPS,), _f32),
            pltpu.VMEM_SHARED((NPAD, H), _f32),
            pltpu.VMEM_SHARED((NPAD,), _f32),
            pltpu.SemaphoreType.DMA,
        ],
    )(functools.partial(_sc_agg_body, rows_linear))


_sc_agg_linear = _make_sc_agg(True)
_sc_agg_gather = _make_sc_agg(False)


def kernel(h, e, edge_index, node2graph, params):
    p = params
    src = edge_index[0].astype(_i32)
    dst = edge_index[1].astype(_i32)
    n2g3 = node2graph.astype(_i32).reshape(NBN, 1, BN)

    h_pad = jnp.pad(h, ((0, 0), (0, 256 - h.shape[1])))
    wpn = jnp.pad(p['gc_pn_w'].T, ((0, 256 - h.shape[1]), (0, 0)))
    bpn = _pack8(p['gc_pn_b'], jnp.zeros((H,), _f32))
    wpe1n = jnp.pad(p['gc_pe1_w'][:, :133].T, ((0, 123), (0, 0)))
    scal1 = _pack8(p['gc_pe2_w'][0, :128],
                   jnp.full((H,), p['gc_pe2_b'][0], _f32))

    hv_new, hn, sd1b = _t1(h_pad, wpn, bpn, wpe1n, scal1)

    ee = _t2(e, p['gc_pe1_w'][:, 133:], _pack8(p['gc_pe1_b']))
    he1 = _sc_he1(hn, src, ee)
    m, se3, mse8 = _t3(he1, p['gc_et_w'].T, _pack8(p['gc_et_b']),
                       _pack8(p['gc_pe2_w'][0, 128:]))
    se = se3.reshape(E)
    sd1 = sd1b[:, 0]
    m16 = jnp.broadcast_to(mse8[0, 0], (16,))
    tbl, den = _sc_agg_linear(dst, sd1, m16, se, m, src)
    den = den.reshape(2, NPAD, 1)

    hs = hv_new
    wih = p['gc_gru_wih'].T
    whh = p['gc_gru_whh'].T
    bih = _pack8w(p['gc_gru_bih'])
    bhh = _pack8w(p['gc_gru_bhh'])
    for i in range(2):
        scal = _pack8(p['l_pe_w'][i][0, :128], p['l_pe_w'][i][0, 128:],
                      jnp.full((H,), p['l_pe_b'][i][0], _f32))
        nf, sdb, ssb, hvp, mssb = _t45(tbl, den, hs, wih, whh, bih, bhh, scal,
                                       p['l_pn_w'][i].T,
                                       _pack8(p['l_pn_b'][i]))
        m16 = jnp.broadcast_to(mssb[0, 0], (16,))
        tbl, den = _sc_agg_gather(dst, sdb[:, 0], m16, ssb[:, 0], hvp, src)
        den = den.reshape(2, NPAD, 1)
        hs = nf
        wih = p['l_gru_wih'][i].T
        whh = p['l_gru_whh'][i].T
        bih = _pack8w(p['l_gru_bih'][i])
        bhh = _pack8w(p['l_gru_bhh'][i])

    clws = _pack8(p['r_cl_w'][0][0, 128:], p['r_cl_w'][1][0, 128:])
    h_out, gf, mhs8 = _t6(tbl, den, hs, wih, whh, bih, bhh, n2g3, clws)

    for t in range(2):
        scal = _pack8(p['r_cl_w'][t][0, :128], p['r_cl_w'][t][0, 128:],
                      jnp.full((H,), p['r_cl_b'][t][0], _f32),
                      jnp.full((H,), mhs8[t, 0], _f32))
        gf = _t78(h_out, n2g3, gf, scal, p['r_pn_w'][t].T,
                  _pack8(p['r_pn_b'][t]),
                  p['r_gru_wih'][t].T, p['r_gru_whh'][t].T,
                  _pack8w(p['r_gru_bih'][t]), _pack8w(p['r_gru_bhh'][t]))
    return gf


# speedup vs baseline: 9.0728x; 1.2257x over previous
import functools

import jax
import jax.numpy as jnp
from jax import lax
from jax.experimental import pallas as pl
from jax.experimental.pallas import tpu as pltpu
from jax.experimental.pallas import tpu_sc as plsc

N = 10000
E = 320000
G = 256
H = 128
BN = 400
NBN = N // BN
BE = 2000
NBE = E // BE
NW = 32
EPW = E // NW
CH = 80
NCH = EPW // CH
NPAD = 10240
RPS = NPAD // 16
ZCH = 128

_f32 = jnp.float32
_i32 = jnp.int32


def _lrelu(x):
    return jnp.where(x >= 0, x, 0.01 * x)


def _elu(x):
    return jnp.where(x > 0, x, jnp.exp(jnp.minimum(x, 0.0)) - 1.0)


def _pack8(*rows):
    rows = list(rows) + [jnp.zeros((128,), _f32)] * (8 - len(rows))
    return jnp.stack(rows)


def _pack8w(*rows):
    rows = list(rows) + [jnp.zeros((384,), _f32)] * (8 - len(rows))
    return jnp.stack(rows)


def _gru(x, hs, wihT, whhT, bih, bhh):
    gi = lax.dot_general(x, wihT, (((1,), (0,)), ((), ())),
                         preferred_element_type=_f32) + bih
    gh = lax.dot_general(hs, whhT, (((1,), (0,)), ((), ())),
                         preferred_element_type=_f32) + bhh
    r = jax.nn.sigmoid(gi[:, 0:128] + gh[:, 0:128])
    z = jax.nn.sigmoid(gi[:, 128:256] + gh[:, 128:256])
    n = jnp.tanh(gi[:, 256:384] + r * gh[:, 256:384])
    return (1.0 - z) * n + z * hs


def _dot(a, b):
    return lax.dot_general(a, b, (((1,), (0,)), ((), ())),
                           preferred_element_type=_f32)


def _t1_body(h_r, wpn_r, bpn_r, wpe1n_r, scal_r, hv_r, hn_r, sd_r):
    x = h_r[...]
    hv = _lrelu(_dot(x, wpn_r[...]) + bpn_r[0:1, :])
    hv_r[...] = hv
    hn_r[...] = _dot(x, wpe1n_r[...])
    sd = jnp.sum(hv * scal_r[0:1, :], axis=1, keepdims=True) + scal_r[1:2, 0:1]
    sd_r[...] = jnp.broadcast_to(sd, (BN, H))


def _t1(h_pad, wpn, bpn, wpe1n, scal):
    return pl.pallas_call(
        _t1_body,
        grid=(NBN,),
        in_specs=[
            pl.BlockSpec((BN, 256), lambda i: (i, 0)),
            pl.BlockSpec((256, H), lambda i: (0, 0)),
            pl.BlockSpec((8, H), lambda i: (0, 0)),
            pl.BlockSpec((256, H), lambda i: (0, 0)),
            pl.BlockSpec((8, H), lambda i: (0, 0)),
        ],
        out_specs=[
            pl.BlockSpec((BN, H), lambda i: (i, 0)),
            pl.BlockSpec((BN, H), lambda i: (i, 0)),
            pl.BlockSpec((BN, H), lambda i: (i, 0)),
        ],
        out_shape=[jax.ShapeDtypeStruct((N, H), _f32)] * 3,
    )(h_pad, wpn, bpn, wpe1n, scal)


def _t2_body(e_r, we_r, be_r, ee_r):
    ee_r[...] = lax.dot_general(e_r[...], we_r[...], (((1,), (1,)), ((), ())),
                                preferred_element_type=_f32) + be_r[0:1, :]


def _t2(e, we, be):
    fe = e.shape[1]
    return pl.pallas_call(
        _t2_body,
        grid=(NBE,),
        in_specs=[
            pl.BlockSpec((BE, fe), lambda i: (i, 0)),
            pl.BlockSpec((H, fe), lambda i: (0, 0)),
            pl.BlockSpec((8, H), lambda i: (0, 0)),
        ],
        out_specs=pl.BlockSpec((BE, H), lambda i: (i, 0)),
        out_shape=jax.ShapeDtypeStruct((E, H), _f32),
    )(e, we, be)


def _t3_body(he_r, wet_r, bet_r, w2b_r, m_r, se_r, mse_r, mx_s):
    i = pl.program_id(0)
    h1 = he_r[...]
    m_r[...] = _dot(h1, wet_r[...]) + bet_r[0:1, :]
    se_row = lax.dot_general(w2b_r[0:1, :], h1, (((1,), (1,)), ((), ())),
                             preferred_element_type=_f32)
    se_r[...] = se_row.reshape(1, 1, BE)
    bmax = jnp.max(se_row)

    @pl.when(i == 0)
    def _():
        mx_s[0] = bmax

    @pl.when(i > 0)
    def _():
        mx_s[0] = jnp.maximum(mx_s[0], bmax)

    @pl.when(i == NBE - 1)
    def _():
        mse_r[...] = jnp.broadcast_to(jnp.maximum(mx_s[0], bmax), (8, H))


def _t3(he1, wet, bet, w2b):
    return pl.pallas_call(
        _t3_body,
        grid=(NBE,),
        in_specs=[
            pl.BlockSpec((BE, H), lambda i: (i, 0)),
            pl.BlockSpec((H, H), lambda i: (0, 0)),
            pl.BlockSpec((8, H), lambda i: (0, 0)),
            pl.BlockSpec((8, H), lambda i: (0, 0)),
        ],
        out_specs=[
            pl.BlockSpec((BE, H), lambda i: (i, 0)),
            pl.BlockSpec((1, 1, BE), lambda i: (i, 0, 0)),
            pl.BlockSpec((8, H), lambda i: (0, 0)),
        ],
        out_shape=[
            jax.ShapeDtypeStruct((E, H), _f32),
            jax.ShapeDtypeStruct((NBE, 1, BE), _f32),
            jax.ShapeDtypeStruct((8, H), _f32),
        ],
        scratch_shapes=[pltpu.SMEM((1,), _f32)],
    )(he1, wet, bet, w2b)


def _t45_body(tbl_r, den_r, hs_r, wihT_r, whhT_r, bih_r, bhh_r, scal_r,
              wpnT_r, bpn_r, nf_r, sd_r, ss_r, hvp_r, mss_r, mx_s):
    i = pl.program_id(0)
    t = tbl_r[...]
    numer = t[0] + t[1]
    d = den_r[...]
    den = d[0] + d[1]
    ctx = _elu(numer / (den + 1e-9))
    nf = jax.nn.relu(_gru(ctx, hs_r[...], wihT_r[...], whhT_r[...],
                          bih_r[0:1, :], bhh_r[0:1, :]))
    nf_r[...] = nf
    sd = jnp.sum(nf * scal_r[0:1, :], axis=1, keepdims=True) + scal_r[2:3, 0:1]
    sd_r[...] = jnp.broadcast_to(sd, (BN, H))
    ss = jnp.sum(nf * scal_r[1:2, :], axis=1, keepdims=True)
    ss_r[...] = jnp.broadcast_to(ss, (BN, H))
    hvp_r[...] = _dot(nf, wpnT_r[...]) + bpn_r[0:1, :]
    bmax = jnp.max(ss)

    @pl.when(i == 0)
    def _():
        mx_s[0] = bmax

    @pl.when(i > 0)
    def _():
        mx_s[0] = jnp.maximum(mx_s[0], bmax)

    @pl.when(i == NBN - 1)
    def _():
        mss_r[...] = jnp.broadcast_to(jnp.maximum(mx_s[0], bmax), (8, H))


def _t45(tbl, den, hs, wihT, whhT, bih, bhh, scal, wpnT, bpn):
    return pl.pallas_call(
        _t45_body,
        grid=(NBN,),
        in_specs=[
            pl.BlockSpec((2, BN, H), lambda i: (0, i, 0)),
            pl.BlockSpec((2, BN, 1), lambda i: (0, i, 0)),
            pl.BlockSpec((BN, H), lambda i: (i, 0)),
            pl.BlockSpec((H, 384), lambda i: (0, 0)),
            pl.BlockSpec((H, 384), lambda i: (0, 0)),
            pl.BlockSpec((8, 384), lambda i: (0, 0)),
            pl.BlockSpec((8, 384), lambda i: (0, 0)),
            pl.BlockSpec((8, H), lambda i: (0, 0)),
            pl.BlockSpec((H, H), lambda i: (0, 0)),
            pl.BlockSpec((8, H), lambda i: (0, 0)),
        ],
        out_specs=[
            pl.BlockSpec((BN, H), lambda i: (i, 0)),
            pl.BlockSpec((BN, H), lambda i: (i, 0)),
            pl.BlockSpec((BN, H), lambda i: (i, 0)),
            pl.BlockSpec((BN, H), lambda i: (i, 0)),
            pl.BlockSpec((8, H), lambda i: (0, 0)),
        ],
        out_shape=[
            jax.ShapeDtypeStruct((N, H), _f32),
            jax.ShapeDtypeStruct((N, H), _f32),
            jax.ShapeDtypeStruct((N, H), _f32),
            jax.ShapeDtypeStruct((N, H), _f32),
            jax.ShapeDtypeStruct((8, H), _f32),
        ],
        scratch_shapes=[pltpu.SMEM((1,), _f32)],
    )(tbl, den, hs, wihT, whhT, bih, bhh, scal, wpnT, bpn)


def _t6_body(tbl_r, den_r, hs_r, wihT_r, whhT_r, bih_r, bhh_r, n2g_r, clws_r,
             h_r, g0_r, mhs_r, mx_s, acc_v):
    i = pl.program_id(0)
    t = tbl_r[...]
    numer = t[0] + t[1]
    d = den_r[...]
    den = d[0] + d[1]
    ctx = _elu(numer / (den + 1e-9))
    nf = jax.nn.relu(_gru(ctx, hs_r[...], wihT_r[...], whhT_r[...],
                          bih_r[0:1, :], bhh_r[0:1, :]))
    h_r[...] = nf
    n2g = n2g_r[0, 0, :]
    oh = (n2g[:, None] == lax.broadcasted_iota(_i32, (BN, G), 1)).astype(_f32)

    @pl.when(i == 0)
    def _():
        acc_v[...] = jnp.zeros((G, H), _f32)

    acc_v[...] += lax.dot_general(oh, nf, (((0,), (0,)), ((), ())),
                                  preferred_element_type=_f32)
    hs0 = jnp.max(jnp.sum(nf * clws_r[0:1, :], axis=1))
    hs1 = jnp.max(jnp.sum(nf * clws_r[1:2, :], axis=1))

    @pl.when(i == 0)
    def _():
        mx_s[0] = hs0
        mx_s[1] = hs1

    @pl.when(i > 0)
    def _():
        mx_s[0] = jnp.maximum(mx_s[0], hs0)
        mx_s[1] = jnp.maximum(mx_s[1], hs1)

    @pl.when(i == NBN - 1)
    def _():
        g0_r[...] = acc_v[...]
        m0 = jnp.maximum(mx_s[0], hs0)
        m1 = jnp.maximum(mx_s[1], hs1)
        mhs_r[...] = jnp.broadcast_to(
            jnp.stack([m0, m1] + [jnp.float32(0.0)] * 6)[:, None], (8, H))


def _t6(tbl, den, hs, wihT, whhT, bih, bhh, n2g3, clws):
    return pl.pallas_call(
        _t6_body,
        grid=(NBN,),
        in_specs=[
            pl.BlockSpec((2, BN, H), lambda i: (0, i, 0)),
            pl.BlockSpec((2, BN, 1), lambda i: (0, i, 0)),
            pl.BlockSpec((BN, H), lambda i: (i, 0)),
            pl.BlockSpec((H, 384), lambda i: (0, 0)),
            pl.BlockSpec((H, 384), lambda i: (0, 0)),
            pl.BlockSpec((8, 384), lambda i: (0, 0)),
            pl.BlockSpec((8, 384), lambda i: (0, 0)),
            pl.BlockSpec((1, 1, BN), lambda i: (i, 0, 0)),
            pl.BlockSpec((8, H), lambda i: (0, 0)),
        ],
        out_specs=[
            pl.BlockSpec((BN, H), lambda i: (i, 0)),
            pl.BlockSpec((G, H), lambda i: (0, 0)),
            pl.BlockSpec((8, H), lambda i: (0, 0)),
        ],
        out_shape=[
            jax.ShapeDtypeStruct((N, H), _f32),
            jax.ShapeDtypeStruct((G, H), _f32),
            jax.ShapeDtypeStruct((8, H), _f32),
        ],
        scratch_shapes=[pltpu.SMEM((2,), _f32), pltpu.VMEM((G, H), _f32)],
    )(tbl, den, hs, wihT, whhT, bih, bhh, n2g3, clws)


def _t78_body(h_r, n2g_r, gf_r, scal_r, wpnT_r, bpn_r, wihT_r, whhT_r,
              bih_r, bhh_r, out_r, gsb_v, nacc_v, dacc_v):
    i = pl.program_id(0)

    @pl.when(i == 0)
    def _():
        gf = gf_r[...]
        gs = jnp.sum(jax.nn.relu(gf) * scal_r[0:1, :], axis=1,
                     keepdims=True) + scal_r[2:3, 0:1]
        gsb_v[...] = jnp.broadcast_to(gs, (G, H))
        nacc_v[...] = jnp.zeros((G, H), _f32)
        dacc_v[...] = jnp.zeros((G, H), _f32)

    hx = h_r[...]
    n2g = n2g_r[0, 0, :]
    oh = (n2g[:, None] == lax.broadcasted_iota(_i32, (BN, G), 1)).astype(_f32)
    gath = lax.dot_general(oh, gsb_v[...], (((1,), (0,)), ((), ())),
                           preferred_element_type=_f32)[:, 0:1]
    hscal = jnp.sum(hx * scal_r[1:2, :], axis=1, keepdims=True)
    z = _lrelu(gath + hscal)
    bd = _lrelu(gath + scal_r[3:4, 0:1])
    ex = jnp.exp(z - bd)
    hvp = _dot(hx, wpnT_r[...]) + bpn_r[0:1, :]
    nacc_v[...] += lax.dot_general(oh, ex * hvp, (((0,), (0,)), ((), ())),
                                   preferred_element_type=_f32)
    dacc_v[...] += lax.dot_general(oh, jnp.broadcast_to(ex, (BN, H)),
                                   (((0,), (0,)), ((), ())),
                                   preferred_element_type=_f32)

    @pl.when(i == NBN - 1)
    def _():
        grep = _elu(nacc_v[...] / (dacc_v[...][:, 0:1] + 1e-9))
        ctx = jax.nn.relu(grep)
        out_r[...] = _gru(ctx, gf_r[...], wihT_r[...], whhT_r[...],
                          bih_r[0:1, :], bhh_r[0:1, :])


def _t78(h_out, n2g3, gf, scal, wpnT, bpn, wihT, whhT, bih, bhh):
    return pl.pallas_call(
        _t78_body,
        grid=(NBN,),
        in_specs=[
            pl.BlockSpec((BN, H), lambda i: (i, 0)),
            pl.BlockSpec((1, 1, BN), lambda i: (i, 0, 0)),
            pl.BlockSpec((G, H), lambda i: (0, 0)),
            pl.BlockSpec((8, H), lambda i: (0, 0)),
            pl.BlockSpec((H, H), lambda i: (0, 0)),
            pl.BlockSpec((8, H), lambda i: (0, 0)),
            pl.BlockSpec((H, 384), lambda i: (0, 0)),
            pl.BlockSpec((H, 384), lambda i: (0, 0)),
            pl.BlockSpec((8, 384), lambda i: (0, 0)),
            pl.BlockSpec((8, 384), lambda i: (0, 0)),
        ],
        out_specs=pl.BlockSpec((G, H), lambda i: (0, 0)),
        out_shape=jax.ShapeDtypeStruct((G, H), _f32),
        scratch_shapes=[pltpu.VMEM((G, H), _f32), pltpu.VMEM((G, H), _f32),
                        pltpu.VMEM((G, H), _f32)],
    )(h_out, n2g3, gf, scal, wpnT, bpn, wihT, whhT, bih, bhh)


_SC_MESH = plsc.VectorSubcoreMesh(core_axis_name="c", subcore_axis_name="s")


@functools.partial(
    pl.kernel,
    out_type=jax.ShapeDtypeStruct((E, H), _f32),
    mesh=_SC_MESH,
    scratch_types=[
        pltpu.VMEM((CH,), _i32),
        pltpu.VMEM((CH, H), _f32),
        pltpu.VMEM((CH, H), _f32),
        pltpu.SemaphoreType.DMA,
    ],
)
def _sc_he1(hn_h, src_h, ee_h, out_h, idx_v, ra_v, rb_v, sem):
    core = lax.axis_index("c")
    sub = lax.axis_index("s")
    wid = core * 16 + sub

    def chunk(ci, carry):
        base = wid * EPW + ci * CH
        pltpu.sync_copy(src_h.at[pl.ds(base, CH)], idx_v)
        cp = pltpu.async_copy(hn_h.at[idx_v], ra_v, sem)
        pltpu.sync_copy(ee_h.at[pl.ds(base, CH)], rb_v)
        cp.wait()

        def row(r, c2):
            for k in range(8):
                a = ra_v[r, pl.ds(k * 16, 16)]
                b = rb_v[r, pl.ds(k * 16, 16)]
                x = a + b
                ra_v[r, pl.ds(k * 16, 16)] = jnp.where(x >= 0, x, 0.01 * x)
            return c2

        lax.fori_loop(0, CH, row, 0)
        pltpu.sync_copy(ra_v, out_h.at[pl.ds(base, CH)])
        return carry

    lax.fori_loop(0, NCH, chunk, 0)


def _sc_agg_body(rows_linear, dst_h, sd_h, mv_h, sev_src_h, rows_src_h,
                 src_h, out_h, outd_h, didx_v, sidx_v, sdv_v, sev_v, exv_v,
                 rows_v, scaled_v, mv_v, zbuf_v, dbuf_v, tbl_sh, den_sh, sem):
    core = lax.axis_index("c")
    sub = lax.axis_index("s")
    wid = core * 16 + sub

    def zrow(r, c):
        for k in range(H // 16):
            zbuf_v[r, pl.ds(k * 16, 16)] = jnp.zeros((16,), _f32)
        return c

    lax.fori_loop(0, ZCH, zrow, 0)
    for i in range(5):
        pltpu.sync_copy(zbuf_v, tbl_sh.at[pl.ds(sub * RPS + i * ZCH, ZCH)])

    def zden(r, c):
        dbuf_v[pl.ds(r * 16, 16)] = jnp.zeros((16,), _f32)
        return c

    lax.fori_loop(0, RPS // 16, zden, 0)
    pltpu.sync_copy(dbuf_v, den_sh.at[pl.ds(sub * RPS, RPS)])
    plsc.subcore_barrier()

    pltpu.sync_copy(mv_h, mv_v)

    def chunk(ci, carry):
        base = wid * EPW + ci * CH
        pltpu.sync_copy(dst_h.at[pl.ds(base, CH)], didx_v)
        g1 = pltpu.async_copy(sd_h.at[didx_v], sdv_v, sem)
        if rows_linear:
            pltpu.sync_copy(sev_src_h.at[pl.ds(base, CH)], sev_v)
            pltpu.sync_copy(rows_src_h.at[pl.ds(base, CH)], rows_v)
            g1.wait()
        else:
            pltpu.sync_copy(src_h.at[pl.ds(base, CH)], sidx_v)
            g2 = pltpu.async_copy(sev_src_h.at[sidx_v], sev_v, sem)
            g3 = pltpu.async_copy(rows_src_h.at[sidx_v], rows_v, sem)
            g1.wait()
            g2.wait()
            g3.wait()
        mv = mv_v[...]
        for g in range(CH // 16):
            s_d = sdv_v[pl.ds(g * 16, 16)]
            s_e = sev_v[pl.ds(g * 16, 16)]
            x = s_d + s_e
            lr = jnp.where(x >= 0, x, 0.01 * x)
            t2 = s_d + mv
            lb = jnp.where(t2 >= 0, t2, 0.01 * t2)
            exv_v[pl.ds(g * 16, 16)] = jnp.exp(lr - lb)
        for g in range(CH // 16):
            exg = exv_v[pl.ds(g * 16, 16)]
            for j in range(16):
                ei = g * 16 + j
                b = jnp.full((16,), exg[j], _f32)
                for k in range(H // 16):
                    scaled_v[ei, pl.ds(k * 16, 16)] = (
                        rows_v[ei, pl.ds(k * 16, 16)] * b)
        pltpu.sync_copy(scaled_v, tbl_sh.at[didx_v], add=True)
        pltpu.sync_copy(exv_v, den_sh.at[didx_v], add=True)
        return carry

    lax.fori_loop(0, NCH, chunk, 0)
    plsc.subcore_barrier()
    for i in range(5):
        pltpu.sync_copy(tbl_sh.at[pl.ds(sub * RPS + i * ZCH, ZCH)], zbuf_v)
        pltpu.sync_copy(zbuf_v, out_h.at[core, pl.ds(sub * RPS + i * ZCH, ZCH)])
    pltpu.sync_copy(den_sh.at[pl.ds(sub * RPS, RPS)], dbuf_v)
    pltpu.sync_copy(dbuf_v, outd_h.at[core, pl.ds(sub * RPS, RPS)])


def _make_sc_agg(rows_linear):
    return functools.partial(
        pl.kernel,
        out_type=[jax.ShapeDtypeStruct((2, NPAD, H), _f32),
                  jax.ShapeDtypeStruct((2, NPAD), _f32)],
        mesh=_SC_MESH,
        scratch_types=[
            pltpu.VMEM((CH,), _i32),
            pltpu.VMEM((CH,), _i32),
            pltpu.VMEM((CH,), _f32),
            pltpu.VMEM((CH,), _f32),
            pltpu.VMEM((CH,), _f32),
            pltpu.VMEM((CH, H), _f32),
            pltpu.VMEM((CH, H), _f32),
            pltpu.VMEM((16,), _f32),
            pltpu.VMEM((ZCH, H), _f32),
            pltpu.VMEM((RPS,), _f32),
            pltpu.VMEM_SHARED((NPAD, H), _f32),
            pltpu.VMEM_SHARED((NPAD,), _f32),
            pltpu.SemaphoreType.DMA,
        ],
    )(functools.partial(_sc_agg_body, rows_linear))


_sc_agg_linear = _make_sc_agg(True)
_sc_agg_gather = _make_sc_agg(False)


def kernel(h, e, edge_index, node2graph, params):
    p = params
    src = edge_index[0].astype(_i32)
    dst = edge_index[1].astype(_i32)
    n2g3 = node2graph.astype(_i32).reshape(NBN, 1, BN)

    h_pad = jnp.pad(h, ((0, 0), (0, 256 - h.shape[1])))
    wpn = jnp.pad(p['gc_pn_w'].T, ((0, 256 - h.shape[1]), (0, 0)))
    bpn = _pack8(p['gc_pn_b'], jnp.zeros((H,), _f32))
    wpe1n = jnp.pad(p['gc_pe1_w'][:, :133].T, ((0, 123), (0, 0)))
    scal1 = _pack8(p['gc_pe2_w'][0, :128],
                   jnp.full((H,), p['gc_pe2_b'][0], _f32))

    hv_new, hn, sd1b = _t1(h_pad, wpn, bpn, wpe1n, scal1)

    ee = _t2(e, p['gc_pe1_w'][:, 133:], _pack8(p['gc_pe1_b']))
    he1 = _sc_he1(hn, src, ee)
    m, se3, mse8 = _t3(he1, p['gc_et_w'].T, _pack8(p['gc_et_b']),
                       _pack8(p['gc_pe2_w'][0, 128:]))
    se = se3.reshape(E)
    sd1 = sd1b[:, 0]
    m16 = jnp.broadcast_to(mse8[0, 0], (16,))
    tbl, den = _sc_agg_linear(dst, sd1, m16, se, m, src)
    den = den.reshape(2, NPAD, 1)

    hs = hv_new
    wih = p['gc_gru_wih'].T
    whh = p['gc_gru_whh'].T
    bih = _pack8w(p['gc_gru_bih'])
    bhh = _pack8w(p['gc_gru_bhh'])
    for i in range(2):
        scal = _pack8(p['l_pe_w'][i][0, :128], p['l_pe_w'][i][0, 128:],
                      jnp.full((H,), p['l_pe_b'][i][0], _f32))
        nf, sdb, ssb, hvp, mssb = _t45(tbl, den, hs, wih, whh, bih, bhh, scal,
                                       p['l_pn_w'][i].T,
                                       _pack8(p['l_pn_b'][i]))
        m16 = jnp.broadcast_to(mssb[0, 0], (16,))
        tbl, den = _sc_agg_gather(dst, sdb[:, 0], m16, ssb[:, 0], hvp, src)
        den = den.reshape(2, NPAD, 1)
        hs = nf
        wih = p['l_gru_wih'][i].T
        whh = p['l_gru_whh'][i].T
        bih = _pack8w(p['l_gru_bih'][i])
        bhh = _pack8w(p['l_gru_bhh'][i])

    clws = _pack8(p['r_cl_w'][0][0, 128:], p['r_cl_w'][1][0, 128:])
    h_out, gf, mhs8 = _t6(tbl, den, hs, wih, whh, bih, bhh, n2g3, clws)

    for t in range(2):
        scal = _pack8(p['r_cl_w'][t][0, :128], p['r_cl_w'][t][0, 128:],
                      jnp.full((H,), p['r_cl_b'][t][0], _f32),
                      jnp.full((H,), mhs8[t, 0], _f32))
        gf = _t78(h_out, n2g3, gf, scal, p['r_pn_w'][t].T,
                  _pack8(p['r_pn_b'][t]),
                  p['r_gru_wih'][t].T, p['r_gru_whh'][t].T,
                  _pack8w(p['r_gru_bih'][t]), _pack8w(p['r_gru_bhh'][t]))
    return gf


# speedup vs baseline: 9.2880x; 1.0237x over previous
import functools

import jax
import jax.numpy as jnp
from jax import lax
from jax.experimental import pallas as pl
from jax.experimental.pallas import tpu as pltpu
from jax.experimental.pallas import tpu_sc as plsc

N = 10000
E = 320000
G = 256
H = 128
BN = 2000
NBN = N // BN
BE = 2000
NBE = E // BE
NW = 32
EPW = E // NW
CH = 80
NCH = EPW // CH
NPAD = 10240
RPS = NPAD // 16
ZCH = 128

_f32 = jnp.float32
_i32 = jnp.int32


def _lrelu(x):
    return jnp.where(x >= 0, x, 0.01 * x)


def _elu(x):
    return jnp.where(x > 0, x, jnp.exp(jnp.minimum(x, 0.0)) - 1.0)


def _pack8(*rows):
    rows = list(rows) + [jnp.zeros((128,), _f32)] * (8 - len(rows))
    return jnp.stack(rows)


def _pack8w(*rows):
    rows = list(rows) + [jnp.zeros((384,), _f32)] * (8 - len(rows))
    return jnp.stack(rows)


def _gru(x, hs, wihT, whhT, bih, bhh):
    gi = lax.dot_general(x, wihT, (((1,), (0,)), ((), ())),
                         preferred_element_type=_f32) + bih
    gh = lax.dot_general(hs, whhT, (((1,), (0,)), ((), ())),
                         preferred_element_type=_f32) + bhh
    r = jax.nn.sigmoid(gi[:, 0:128] + gh[:, 0:128])
    z = jax.nn.sigmoid(gi[:, 128:256] + gh[:, 128:256])
    n = jnp.tanh(gi[:, 256:384] + r * gh[:, 256:384])
    return (1.0 - z) * n + z * hs


def _dot(a, b):
    return lax.dot_general(a, b, (((1,), (0,)), ((), ())),
                           preferred_element_type=_f32)


def _t1_body(h_r, wpn_r, bpn_r, wpe1n_r, scal_r, hv_r, hn_r, sd_r):
    x = h_r[...]
    hv = _lrelu(_dot(x, wpn_r[...]) + bpn_r[0:1, :])
    hv_r[...] = hv
    hn_r[...] = _dot(x, wpe1n_r[...])
    sd = jnp.sum(hv * scal_r[0:1, :], axis=1, keepdims=True) + scal_r[1:2, 0:1]
    sd_r[...] = jnp.broadcast_to(sd, (BN, H))


def _t1(h_pad, wpn, bpn, wpe1n, scal):
    return pl.pallas_call(
        _t1_body,
        grid=(NBN,),
        in_specs=[
            pl.BlockSpec((BN, 256), lambda i: (i, 0)),
            pl.BlockSpec((256, H), lambda i: (0, 0)),
            pl.BlockSpec((8, H), lambda i: (0, 0)),
            pl.BlockSpec((256, H), lambda i: (0, 0)),
            pl.BlockSpec((8, H), lambda i: (0, 0)),
        ],
        out_specs=[
            pl.BlockSpec((BN, H), lambda i: (i, 0)),
            pl.BlockSpec((BN, H), lambda i: (i, 0)),
            pl.BlockSpec((BN, H), lambda i: (i, 0)),
        ],
        out_shape=[jax.ShapeDtypeStruct((N, H), _f32)] * 3,
    )(h_pad, wpn, bpn, wpe1n, scal)


def _t2_body(e_r, we_r, be_r, ee_r):
    ee_r[...] = lax.dot_general(e_r[...], we_r[...], (((1,), (1,)), ((), ())),
                                preferred_element_type=_f32) + be_r[0:1, :]


def _t2(e, we, be):
    fe = e.shape[1]
    return pl.pallas_call(
        _t2_body,
        grid=(NBE,),
        in_specs=[
            pl.BlockSpec((BE, fe), lambda i: (i, 0)),
            pl.BlockSpec((H, fe), lambda i: (0, 0)),
            pl.BlockSpec((8, H), lambda i: (0, 0)),
        ],
        out_specs=pl.BlockSpec((BE, H), lambda i: (i, 0)),
        out_shape=jax.ShapeDtypeStruct((E, H), _f32),
    )(e, we, be)


def _t3_body(he_r, wet_r, bet_r, w2b_r, m_r, se_r, mse_r, mx_s):
    i = pl.program_id(0)
    h1 = he_r[...]
    m_r[...] = _dot(h1, wet_r[...]) + bet_r[0:1, :]
    se_row = lax.dot_general(w2b_r[0:1, :], h1, (((1,), (1,)), ((), ())),
                             preferred_element_type=_f32)
    se_r[...] = se_row.reshape(1, 1, BE)
    bmax = jnp.max(se_row)

    @pl.when(i == 0)
    def _():
        mx_s[0] = bmax

    @pl.when(i > 0)
    def _():
        mx_s[0] = jnp.maximum(mx_s[0], bmax)

    @pl.when(i == NBE - 1)
    def _():
        mse_r[...] = jnp.broadcast_to(jnp.maximum(mx_s[0], bmax), (8, H))


def _t3(he1, wet, bet, w2b):
    return pl.pallas_call(
        _t3_body,
        grid=(NBE,),
        in_specs=[
            pl.BlockSpec((BE, H), lambda i: (i, 0)),
            pl.BlockSpec((H, H), lambda i: (0, 0)),
            pl.BlockSpec((8, H), lambda i: (0, 0)),
            pl.BlockSpec((8, H), lambda i: (0, 0)),
        ],
        out_specs=[
            pl.BlockSpec((BE, H), lambda i: (i, 0)),
            pl.BlockSpec((1, 1, BE), lambda i: (i, 0, 0)),
            pl.BlockSpec((8, H), lambda i: (0, 0)),
        ],
        out_shape=[
            jax.ShapeDtypeStruct((E, H), _f32),
            jax.ShapeDtypeStruct((NBE, 1, BE), _f32),
            jax.ShapeDtypeStruct((8, H), _f32),
        ],
        scratch_shapes=[pltpu.SMEM((1,), _f32)],
    )(he1, wet, bet, w2b)


def _t45_body(tbl_r, den_r, hs_r, wihT_r, whhT_r, bih_r, bhh_r, scal_r,
              wpnT_r, bpn_r, nf_r, sd_r, ss_r, hvp_r, mss_r, mx_s):
    i = pl.program_id(0)
    t = tbl_r[...]
    numer = t[0] + t[1]
    d = den_r[...]
    den = d[0] + d[1]
    ctx = _elu(numer / (den + 1e-9))
    nf = jax.nn.relu(_gru(ctx, hs_r[...], wihT_r[...], whhT_r[...],
                          bih_r[0:1, :], bhh_r[0:1, :]))
    nf_r[...] = nf
    sd = jnp.sum(nf * scal_r[0:1, :], axis=1, keepdims=True) + scal_r[2:3, 0:1]
    sd_r[...] = jnp.broadcast_to(sd, (BN, H))
    ss = jnp.sum(nf * scal_r[1:2, :], axis=1, keepdims=True)
    ss_r[...] = jnp.broadcast_to(ss, (BN, H))
    hvp_r[...] = _dot(nf, wpnT_r[...]) + bpn_r[0:1, :]
    bmax = jnp.max(ss)

    @pl.when(i == 0)
    def _():
        mx_s[0] = bmax

    @pl.when(i > 0)
    def _():
        mx_s[0] = jnp.maximum(mx_s[0], bmax)

    @pl.when(i == NBN - 1)
    def _():
        mss_r[...] = jnp.broadcast_to(jnp.maximum(mx_s[0], bmax), (8, H))


def _t45(tbl, den, hs, wihT, whhT, bih, bhh, scal, wpnT, bpn):
    return pl.pallas_call(
        _t45_body,
        grid=(NBN,),
        in_specs=[
            pl.BlockSpec((2, BN, H), lambda i: (0, i, 0)),
            pl.BlockSpec((2, BN, 1), lambda i: (0, i, 0)),
            pl.BlockSpec((BN, H), lambda i: (i, 0)),
            pl.BlockSpec((H, 384), lambda i: (0, 0)),
            pl.BlockSpec((H, 384), lambda i: (0, 0)),
            pl.BlockSpec((8, 384), lambda i: (0, 0)),
            pl.BlockSpec((8, 384), lambda i: (0, 0)),
            pl.BlockSpec((8, H), lambda i: (0, 0)),
            pl.BlockSpec((H, H), lambda i: (0, 0)),
            pl.BlockSpec((8, H), lambda i: (0, 0)),
        ],
        out_specs=[
            pl.BlockSpec((BN, H), lambda i: (i, 0)),
            pl.BlockSpec((BN, H), lambda i: (i, 0)),
            pl.BlockSpec((BN, H), lambda i: (i, 0)),
            pl.BlockSpec((BN, H), lambda i: (i, 0)),
            pl.BlockSpec((8, H), lambda i: (0, 0)),
        ],
        out_shape=[
            jax.ShapeDtypeStruct((N, H), _f32),
            jax.ShapeDtypeStruct((N, H), _f32),
            jax.ShapeDtypeStruct((N, H), _f32),
            jax.ShapeDtypeStruct((N, H), _f32),
            jax.ShapeDtypeStruct((8, H), _f32),
        ],
        scratch_shapes=[pltpu.SMEM((1,), _f32)],
    )(tbl, den, hs, wihT, whhT, bih, bhh, scal, wpnT, bpn)


def _t6_body(tbl_r, den_r, hs_r, wihT_r, whhT_r, bih_r, bhh_r, n2g_r, clws_r,
             h_r, g0_r, mhs_r, mx_s, acc_v):
    i = pl.program_id(0)
    t = tbl_r[...]
    numer = t[0] + t[1]
    d = den_r[...]
    den = d[0] + d[1]
    ctx = _elu(numer / (den + 1e-9))
    nf = jax.nn.relu(_gru(ctx, hs_r[...], wihT_r[...], whhT_r[...],
                          bih_r[0:1, :], bhh_r[0:1, :]))
    h_r[...] = nf
    n2g = n2g_r[0, 0, :]
    oh = (n2g[:, None] == lax.broadcasted_iota(_i32, (BN, G), 1)).astype(_f32)

    @pl.when(i == 0)
    def _():
        acc_v[...] = jnp.zeros((G, H), _f32)

    acc_v[...] += lax.dot_general(oh, nf, (((0,), (0,)), ((), ())),
                                  preferred_element_type=_f32)
    hs0 = jnp.max(jnp.sum(nf * clws_r[0:1, :], axis=1))
    hs1 = jnp.max(jnp.sum(nf * clws_r[1:2, :], axis=1))

    @pl.when(i == 0)
    def _():
        mx_s[0] = hs0
        mx_s[1] = hs1

    @pl.when(i > 0)
    def _():
        mx_s[0] = jnp.maximum(mx_s[0], hs0)
        mx_s[1] = jnp.maximum(mx_s[1], hs1)

    @pl.when(i == NBN - 1)
    def _():
        g0_r[...] = acc_v[...]
        m0 = jnp.maximum(mx_s[0], hs0)
        m1 = jnp.maximum(mx_s[1], hs1)
        mhs_r[...] = jnp.broadcast_to(
            jnp.stack([m0, m1] + [jnp.float32(0.0)] * 6)[:, None], (8, H))


def _t6(tbl, den, hs, wihT, whhT, bih, bhh, n2g3, clws):
    return pl.pallas_call(
        _t6_body,
        grid=(NBN,),
        in_specs=[
            pl.BlockSpec((2, BN, H), lambda i: (0, i, 0)),
            pl.BlockSpec((2, BN, 1), lambda i: (0, i, 0)),
            pl.BlockSpec((BN, H), lambda i: (i, 0)),
            pl.BlockSpec((H, 384), lambda i: (0, 0)),
            pl.BlockSpec((H, 384), lambda i: (0, 0)),
            pl.BlockSpec((8, 384), lambda i: (0, 0)),
            pl.BlockSpec((8, 384), lambda i: (0, 0)),
            pl.BlockSpec((1, 1, BN), lambda i: (i, 0, 0)),
            pl.BlockSpec((8, H), lambda i: (0, 0)),
        ],
        out_specs=[
            pl.BlockSpec((BN, H), lambda i: (i, 0)),
            pl.BlockSpec((G, H), lambda i: (0, 0)),
            pl.BlockSpec((8, H), lambda i: (0, 0)),
        ],
        out_shape=[
            jax.ShapeDtypeStruct((N, H), _f32),
            jax.ShapeDtypeStruct((G, H), _f32),
            jax.ShapeDtypeStruct((8, H), _f32),
        ],
        scratch_shapes=[pltpu.SMEM((2,), _f32), pltpu.VMEM((G, H), _f32)],
    )(tbl, den, hs, wihT, whhT, bih, bhh, n2g3, clws)


def _t78_body(h_r, n2g_r, gf_r, scal_r, wpnT_r, bpn_r, wihT_r, whhT_r,
              bih_r, bhh_r, out_r, gsb_v, nacc_v, dacc_v):
    i = pl.program_id(0)

    @pl.when(i == 0)
    def _():
        gf = gf_r[...]
        gs = jnp.sum(jax.nn.relu(gf) * scal_r[0:1, :], axis=1,
                     keepdims=True) + scal_r[2:3, 0:1]
        gsb_v[...] = jnp.broadcast_to(gs, (G, H))
        nacc_v[...] = jnp.zeros((G, H), _f32)
        dacc_v[...] = jnp.zeros((G, H), _f32)

    hx = h_r[...]
    n2g = n2g_r[0, 0, :]
    oh = (n2g[:, None] == lax.broadcasted_iota(_i32, (BN, G), 1)).astype(_f32)
    gath = lax.dot_general(oh, gsb_v[...], (((1,), (0,)), ((), ())),
                           preferred_element_type=_f32)[:, 0:1]
    hscal = jnp.sum(hx * scal_r[1:2, :], axis=1, keepdims=True)
    z = _lrelu(gath + hscal)
    bd = _lrelu(gath + scal_r[3:4, 0:1])
    ex = jnp.exp(z - bd)
    hvp = _dot(hx, wpnT_r[...]) + bpn_r[0:1, :]
    nacc_v[...] += lax.dot_general(oh, ex * hvp, (((0,), (0,)), ((), ())),
                                   preferred_element_type=_f32)
    dacc_v[...] += lax.dot_general(oh, jnp.broadcast_to(ex, (BN, H)),
                                   (((0,), (0,)), ((), ())),
                                   preferred_element_type=_f32)

    @pl.when(i == NBN - 1)
    def _():
        grep = _elu(nacc_v[...] / (dacc_v[...][:, 0:1] + 1e-9))
        ctx = jax.nn.relu(grep)
        out_r[...] = _gru(ctx, gf_r[...], wihT_r[...], whhT_r[...],
                          bih_r[0:1, :], bhh_r[0:1, :])


def _t78(h_out, n2g3, gf, scal, wpnT, bpn, wihT, whhT, bih, bhh):
    return pl.pallas_call(
        _t78_body,
        grid=(NBN,),
        in_specs=[
            pl.BlockSpec((BN, H), lambda i: (i, 0)),
            pl.BlockSpec((1, 1, BN), lambda i: (i, 0, 0)),
            pl.BlockSpec((G, H), lambda i: (0, 0)),
            pl.BlockSpec((8, H), lambda i: (0, 0)),
            pl.BlockSpec((H, H), lambda i: (0, 0)),
            pl.BlockSpec((8, H), lambda i: (0, 0)),
            pl.BlockSpec((H, 384), lambda i: (0, 0)),
            pl.BlockSpec((H, 384), lambda i: (0, 0)),
            pl.BlockSpec((8, 384), lambda i: (0, 0)),
            pl.BlockSpec((8, 384), lambda i: (0, 0)),
        ],
        out_specs=pl.BlockSpec((G, H), lambda i: (0, 0)),
        out_shape=jax.ShapeDtypeStruct((G, H), _f32),
        scratch_shapes=[pltpu.VMEM((G, H), _f32), pltpu.VMEM((G, H), _f32),
                        pltpu.VMEM((G, H), _f32)],
    )(h_out, n2g3, gf, scal, wpnT, bpn, wihT, whhT, bih, bhh)


_SC_MESH = plsc.VectorSubcoreMesh(core_axis_name="c", subcore_axis_name="s")


@functools.partial(
    pl.kernel,
    out_type=jax.ShapeDtypeStruct((E, H), _f32),
    mesh=_SC_MESH,
    scratch_types=[
        pltpu.VMEM((CH,), _i32),
        pltpu.VMEM((CH, H), _f32),
        pltpu.VMEM((CH, H), _f32),
        pltpu.SemaphoreType.DMA,
    ],
)
def _sc_he1(hn_h, src_h, ee_h, out_h, idx_v, ra_v, rb_v, sem):
    core = lax.axis_index("c")
    sub = lax.axis_index("s")
    wid = core * 16 + sub

    def chunk(ci, carry):
        base = wid * EPW + ci * CH
        pltpu.sync_copy(src_h.at[pl.ds(base, CH)], idx_v)
        cp = pltpu.async_copy(hn_h.at[idx_v], ra_v, sem)
        pltpu.sync_copy(ee_h.at[pl.ds(base, CH)], rb_v)
        cp.wait()

        def row(r, c2):
            for k in range(8):
                a = ra_v[r, pl.ds(k * 16, 16)]
                b = rb_v[r, pl.ds(k * 16, 16)]
                x = a + b
                ra_v[r, pl.ds(k * 16, 16)] = jnp.where(x >= 0, x, 0.01 * x)
            return c2

        lax.fori_loop(0, CH, row, 0)
        pltpu.sync_copy(ra_v, out_h.at[pl.ds(base, CH)])
        return carry

    lax.fori_loop(0, NCH, chunk, 0)


def _sc_agg_body(rows_linear, dst_h, sd_h, mv_h, sev_src_h, rows_src_h,
                 src_h, out_h, outd_h, didx_v, sidx_v, sdv_v, sev_v, exv_v,
                 rows_v, scaled_v, mv_v, zbuf_v, dbuf_v, tbl_sh, den_sh, sem):
    core = lax.axis_index("c")
    sub = lax.axis_index("s")
    wid = core * 16 + sub

    def zrow(r, c):
        for k in range(H // 16):
            zbuf_v[r, pl.ds(k * 16, 16)] = jnp.zeros((16,), _f32)
        return c

    lax.fori_loop(0, ZCH, zrow, 0)
    for i in range(5):
        pltpu.sync_copy(zbuf_v, tbl_sh.at[pl.ds(sub * RPS + i * ZCH, ZCH)])

    def zden(r, c):
        dbuf_v[pl.ds(r * 16, 16)] = jnp.zeros((16,), _f32)
        return c

    lax.fori_loop(0, RPS // 16, zden, 0)
    pltpu.sync_copy(dbuf_v, den_sh.at[pl.ds(sub * RPS, RPS)])
    plsc.subcore_barrier()

    pltpu.sync_copy(mv_h, mv_v)

    def chunk(ci, carry):
        base = wid * EPW + ci * CH
        pltpu.sync_copy(dst_h.at[pl.ds(base, CH)], didx_v)
        g1 = pltpu.async_copy(sd_h.at[didx_v], sdv_v, sem)
        if rows_linear:
            pltpu.sync_copy(sev_src_h.at[pl.ds(base, CH)], sev_v)
            pltpu.sync_copy(rows_src_h.at[pl.ds(base, CH)], rows_v)
            g1.wait()
        else:
            pltpu.sync_copy(src_h.at[pl.ds(base, CH)], sidx_v)
            g2 = pltpu.async_copy(sev_src_h.at[sidx_v], sev_v, sem)
            g3 = pltpu.async_copy(rows_src_h.at[sidx_v], rows_v, sem)
            g1.wait()
            g2.wait()
            g3.wait()
        mv = mv_v[...]
        for g in range(CH // 16):
            s_d = sdv_v[pl.ds(g * 16, 16)]
            s_e = sev_v[pl.ds(g * 16, 16)]
            x = s_d + s_e
            lr = jnp.where(x >= 0, x, 0.01 * x)
            t2 = s_d + mv
            lb = jnp.where(t2 >= 0, t2, 0.01 * t2)
            exv_v[pl.ds(g * 16, 16)] = jnp.exp(lr - lb)
        for g in range(CH // 16):
            exg = exv_v[pl.ds(g * 16, 16)]
            for j in range(16):
                ei = g * 16 + j
                b = jnp.full((16,), exg[j], _f32)
                for k in range(H // 16):
                    scaled_v[ei, pl.ds(k * 16, 16)] = (
                        rows_v[ei, pl.ds(k * 16, 16)] * b)
        pltpu.sync_copy(scaled_v, tbl_sh.at[didx_v], add=True)
        pltpu.sync_copy(exv_v, den_sh.at[didx_v], add=True)
        return carry

    lax.fori_loop(0, NCH, chunk, 0)
    plsc.subcore_barrier()
    for i in range(5):
        pltpu.sync_copy(tbl_sh.at[pl.ds(sub * RPS + i * ZCH, ZCH)], zbuf_v)
        pltpu.sync_copy(zbuf_v, out_h.at[core, pl.ds(sub * RPS + i * ZCH, ZCH)])
    pltpu.sync_copy(den_sh.at[pl.ds(sub * RPS, RPS)], dbuf_v)
    pltpu.sync_copy(dbuf_v, outd_h.at[core, pl.ds(sub * RPS, RPS)])


def _make_sc_agg(rows_linear):
    return functools.partial(
        pl.kernel,
        out_type=[jax.ShapeDtypeStruct((2, NPAD, H), _f32),
                  jax.ShapeDtypeStruct((2, NPAD), _f32)],
        mesh=_SC_MESH,
        scratch_types=[
            pltpu.VMEM((CH,), _i32),
            pltpu.VMEM((CH,), _i32),
            pltpu.VMEM((CH,), _f32),
            pltpu.VMEM((CH,), _f32),
            pltpu.VMEM((CH,), _f32),
            pltpu.VMEM((CH, H), _f32),
            pltpu.VMEM((CH, H), _f32),
            pltpu.VMEM((16,), _f32),
            pltpu.VMEM((ZCH, H), _f32),
            pltpu.VMEM((RPS,), _f32),
            pltpu.VMEM_SHARED((NPAD, H), _f32),
            pltpu.VMEM_SHARED((NPAD,), _f32),
            pltpu.SemaphoreType.DMA,
        ],
    )(functools.partial(_sc_agg_body, rows_linear))


_sc_agg_linear = _make_sc_agg(True)
_sc_agg_gather = _make_sc_agg(False)


def kernel(h, e, edge_index, node2graph, params):
    p = params
    src = edge_index[0].astype(_i32)
    dst = edge_index[1].astype(_i32)
    n2g3 = node2graph.astype(_i32).reshape(NBN, 1, BN)

    h_pad = jnp.pad(h, ((0, 0), (0, 256 - h.shape[1])))
    wpn = jnp.pad(p['gc_pn_w'].T, ((0, 256 - h.shape[1]), (0, 0)))
    bpn = _pack8(p['gc_pn_b'], jnp.zeros((H,), _f32))
    wpe1n = jnp.pad(p['gc_pe1_w'][:, :133].T, ((0, 123), (0, 0)))
    scal1 = _pack8(p['gc_pe2_w'][0, :128],
                   jnp.full((H,), p['gc_pe2_b'][0], _f32))

    hv_new, hn, sd1b = _t1(h_pad, wpn, bpn, wpe1n, scal1)

    ee = _t2(e, p['gc_pe1_w'][:, 133:], _pack8(p['gc_pe1_b']))
    he1 = _sc_he1(hn, src, ee)
    m, se3, mse8 = _t3(he1, p['gc_et_w'].T, _pack8(p['gc_et_b']),
                       _pack8(p['gc_pe2_w'][0, 128:]))
    se = se3.reshape(E)
    sd1 = sd1b[:, 0]
    m16 = jnp.broadcast_to(mse8[0, 0], (16,))
    tbl, den = _sc_agg_linear(dst, sd1, m16, se, m, src)
    den = den.reshape(2, NPAD, 1)

    hs = hv_new
    wih = p['gc_gru_wih'].T
    whh = p['gc_gru_whh'].T
    bih = _pack8w(p['gc_gru_bih'])
    bhh = _pack8w(p['gc_gru_bhh'])
    for i in range(2):
        scal = _pack8(p['l_pe_w'][i][0, :128], p['l_pe_w'][i][0, 128:],
                      jnp.full((H,), p['l_pe_b'][i][0], _f32))
        nf, sdb, ssb, hvp, mssb = _t45(tbl, den, hs, wih, whh, bih, bhh, scal,
                                       p['l_pn_w'][i].T,
                                       _pack8(p['l_pn_b'][i]))
        m16 = jnp.broadcast_to(mssb[0, 0], (16,))
        tbl, den = _sc_agg_gather(dst, sdb[:, 0], m16, ssb[:, 0], hvp, src)
        den = den.reshape(2, NPAD, 1)
        hs = nf
        wih = p['l_gru_wih'][i].T
        whh = p['l_gru_whh'][i].T
        bih = _pack8w(p['l_gru_bih'][i])
        bhh = _pack8w(p['l_gru_bhh'][i])

    clws = _pack8(p['r_cl_w'][0][0, 128:], p['r_cl_w'][1][0, 128:])
    h_out, gf, mhs8 = _t6(tbl, den, hs, wih, whh, bih, bhh, n2g3, clws)

    for t in range(2):
        scal = _pack8(p['r_cl_w'][t][0, :128], p['r_cl_w'][t][0, 128:],
                      jnp.full((H,), p['r_cl_b'][t][0], _f32),
                      jnp.full((H,), mhs8[t, 0], _f32))
        gf = _t78(h_out, n2g3, gf, scal, p['r_pn_w'][t].T,
                  _pack8(p['r_pn_b'][t]),
                  p['r_gru_wih'][t].T, p['r_gru_whh'][t].T,
                  _pack8w(p['r_gru_bih'][t]), _pack8w(p['r_gru_bhh'][t]))
    return gf


# speedup vs baseline: 11.3204x; 1.2188x over previous
import functools

import jax
import jax.numpy as jnp
from jax import lax
from jax.experimental import pallas as pl
from jax.experimental.pallas import tpu as pltpu
from jax.experimental.pallas import tpu_sc as plsc

N = 10000
E = 320000
G = 256
H = 128
BN = 2000
NBN = N // BN
BE = 2000
NBE = E // BE
NW = 32
EPW = E // NW
CH = 80
NCH = EPW // CH
NPAD = 10240
RPS = NPAD // 16
ZCH = 128

_f32 = jnp.float32
_i32 = jnp.int32


def _lrelu(x):
    return jnp.where(x >= 0, x, 0.01 * x)


def _elu(x):
    return jnp.where(x > 0, x, jnp.exp(jnp.minimum(x, 0.0)) - 1.0)


def _pack8(*rows):
    rows = list(rows) + [jnp.zeros((128,), _f32)] * (8 - len(rows))
    return jnp.stack(rows)


def _pack8w(*rows):
    rows = list(rows) + [jnp.zeros((384,), _f32)] * (8 - len(rows))
    return jnp.stack(rows)


def _gru(x, hs, wihT, whhT, bih, bhh):
    gi = lax.dot_general(x, wihT, (((1,), (0,)), ((), ())),
                         preferred_element_type=_f32) + bih
    gh = lax.dot_general(hs, whhT, (((1,), (0,)), ((), ())),
                         preferred_element_type=_f32) + bhh
    r = jax.nn.sigmoid(gi[:, 0:128] + gh[:, 0:128])
    z = jax.nn.sigmoid(gi[:, 128:256] + gh[:, 128:256])
    n = jnp.tanh(gi[:, 256:384] + r * gh[:, 256:384])
    return (1.0 - z) * n + z * hs


def _dot(a, b):
    return lax.dot_general(a, b, (((1,), (0,)), ((), ())),
                           preferred_element_type=_f32)


def _t1_body(h_r, wpn_r, bpn_r, wpe1n_r, scal_r, hv_r, hn_r, sd_r):
    x = h_r[...]
    hv = _lrelu(_dot(x, wpn_r[...]) + bpn_r[0:1, :])
    hv_r[...] = hv
    hn_r[...] = _dot(x, wpe1n_r[...])
    sd = jnp.sum(hv * scal_r[0:1, :], axis=1, keepdims=True) + scal_r[1:2, 0:1]
    sd_r[...] = jnp.broadcast_to(sd, (BN, H))


def _t1(h_pad, wpn, bpn, wpe1n, scal):
    return pl.pallas_call(
        _t1_body,
        grid=(NBN,),
        in_specs=[
            pl.BlockSpec((BN, 256), lambda i: (i, 0)),
            pl.BlockSpec((256, H), lambda i: (0, 0)),
            pl.BlockSpec((8, H), lambda i: (0, 0)),
            pl.BlockSpec((256, H), lambda i: (0, 0)),
            pl.BlockSpec((8, H), lambda i: (0, 0)),
        ],
        out_specs=[
            pl.BlockSpec((BN, H), lambda i: (i, 0)),
            pl.BlockSpec((BN, H), lambda i: (i, 0)),
            pl.BlockSpec((BN, H), lambda i: (i, 0)),
        ],
        out_shape=[jax.ShapeDtypeStruct((N, H), _f32)] * 3,
    )(h_pad, wpn, bpn, wpe1n, scal)


def _t2_body(e_r, we_r, be_r, ee_r):
    ee_r[...] = lax.dot_general(e_r[...], we_r[...], (((1,), (1,)), ((), ())),
                                preferred_element_type=_f32) + be_r[0:1, :]


def _t2(e, we, be):
    fe = e.shape[1]
    return pl.pallas_call(
        _t2_body,
        grid=(NBE,),
        in_specs=[
            pl.BlockSpec((BE, fe), lambda i: (i, 0)),
            pl.BlockSpec((H, fe), lambda i: (0, 0)),
            pl.BlockSpec((8, H), lambda i: (0, 0)),
        ],
        out_specs=pl.BlockSpec((BE, H), lambda i: (i, 0)),
        out_shape=jax.ShapeDtypeStruct((E, H), _f32),
    )(e, we, be)


def _t3_body(he_r, wet_r, bet_r, w2b_r, m_r, se_r, mse_r, mx_s):
    i = pl.program_id(0)
    h1 = he_r[...]
    m_r[...] = _dot(h1, wet_r[...]) + bet_r[0:1, :]
    se_row = lax.dot_general(w2b_r[0:1, :], h1, (((1,), (1,)), ((), ())),
                             preferred_element_type=_f32)
    se_r[...] = se_row.reshape(1, 1, BE)
    bmax = jnp.max(se_row)

    @pl.when(i == 0)
    def _():
        mx_s[0] = bmax

    @pl.when(i > 0)
    def _():
        mx_s[0] = jnp.maximum(mx_s[0], bmax)

    @pl.when(i == NBE - 1)
    def _():
        mse_r[...] = jnp.broadcast_to(jnp.maximum(mx_s[0], bmax), (8, H))


def _t3(he1, wet, bet, w2b):
    return pl.pallas_call(
        _t3_body,
        grid=(NBE,),
        in_specs=[
            pl.BlockSpec((BE, H), lambda i: (i, 0)),
            pl.BlockSpec((H, H), lambda i: (0, 0)),
            pl.BlockSpec((8, H), lambda i: (0, 0)),
            pl.BlockSpec((8, H), lambda i: (0, 0)),
        ],
        out_specs=[
            pl.BlockSpec((BE, H), lambda i: (i, 0)),
            pl.BlockSpec((1, 1, BE), lambda i: (i, 0, 0)),
            pl.BlockSpec((8, H), lambda i: (0, 0)),
        ],
        out_shape=[
            jax.ShapeDtypeStruct((E, H), _f32),
            jax.ShapeDtypeStruct((NBE, 1, BE), _f32),
            jax.ShapeDtypeStruct((8, H), _f32),
        ],
        scratch_shapes=[pltpu.SMEM((1,), _f32)],
    )(he1, wet, bet, w2b)


def _t45_body(tbl_r, den_r, hs_r, wihT_r, whhT_r, bih_r, bhh_r, scal_r,
              wpnT_r, bpn_r, nf_r, sd_r, ss_r, hvp_r, mss_r, mx_s):
    i = pl.program_id(0)
    t = tbl_r[...]
    numer = t[0] + t[1]
    d = den_r[...]
    den = d[0] + d[1]
    ctx = _elu(numer / (den + 1e-9))
    nf = jax.nn.relu(_gru(ctx, hs_r[...], wihT_r[...], whhT_r[...],
                          bih_r[0:1, :], bhh_r[0:1, :]))
    nf_r[...] = nf
    sd = jnp.sum(nf * scal_r[0:1, :], axis=1, keepdims=True) + scal_r[2:3, 0:1]
    sd_r[...] = jnp.broadcast_to(sd, (BN, H))
    ss = jnp.sum(nf * scal_r[1:2, :], axis=1, keepdims=True)
    ss_r[...] = jnp.broadcast_to(ss, (BN, H))
    hvp_r[...] = _dot(nf, wpnT_r[...]) + bpn_r[0:1, :]
    bmax = jnp.max(ss)

    @pl.when(i == 0)
    def _():
        mx_s[0] = bmax

    @pl.when(i > 0)
    def _():
        mx_s[0] = jnp.maximum(mx_s[0], bmax)

    @pl.when(i == NBN - 1)
    def _():
        mss_r[...] = jnp.broadcast_to(jnp.maximum(mx_s[0], bmax), (8, H))


def _t45(tbl, den, hs, wihT, whhT, bih, bhh, scal, wpnT, bpn):
    return pl.pallas_call(
        _t45_body,
        grid=(NBN,),
        in_specs=[
            pl.BlockSpec((2, BN, H), lambda i: (0, i, 0)),
            pl.BlockSpec((2, BN, 1), lambda i: (0, i, 0)),
            pl.BlockSpec((BN, H), lambda i: (i, 0)),
            pl.BlockSpec((H, 384), lambda i: (0, 0)),
            pl.BlockSpec((H, 384), lambda i: (0, 0)),
            pl.BlockSpec((8, 384), lambda i: (0, 0)),
            pl.BlockSpec((8, 384), lambda i: (0, 0)),
            pl.BlockSpec((8, H), lambda i: (0, 0)),
            pl.BlockSpec((H, H), lambda i: (0, 0)),
            pl.BlockSpec((8, H), lambda i: (0, 0)),
        ],
        out_specs=[
            pl.BlockSpec((BN, H), lambda i: (i, 0)),
            pl.BlockSpec((BN, H), lambda i: (i, 0)),
            pl.BlockSpec((BN, H), lambda i: (i, 0)),
            pl.BlockSpec((BN, H), lambda i: (i, 0)),
            pl.BlockSpec((8, H), lambda i: (0, 0)),
        ],
        out_shape=[
            jax.ShapeDtypeStruct((N, H), _f32),
            jax.ShapeDtypeStruct((N, H), _f32),
            jax.ShapeDtypeStruct((N, H), _f32),
            jax.ShapeDtypeStruct((N, H), _f32),
            jax.ShapeDtypeStruct((8, H), _f32),
        ],
        scratch_shapes=[pltpu.SMEM((1,), _f32)],
    )(tbl, den, hs, wihT, whhT, bih, bhh, scal, wpnT, bpn)


def _t6_body(tbl_r, den_r, hs_r, wihT_r, whhT_r, bih_r, bhh_r, n2g_r, clws_r,
             h_r, g0_r, mhs_r, mx_s, acc_v):
    i = pl.program_id(0)
    t = tbl_r[...]
    numer = t[0] + t[1]
    d = den_r[...]
    den = d[0] + d[1]
    ctx = _elu(numer / (den + 1e-9))
    nf = jax.nn.relu(_gru(ctx, hs_r[...], wihT_r[...], whhT_r[...],
                          bih_r[0:1, :], bhh_r[0:1, :]))
    h_r[...] = nf
    n2g = n2g_r[0, 0, :]
    oh = (n2g[:, None] == lax.broadcasted_iota(_i32, (BN, G), 1)).astype(_f32)

    @pl.when(i == 0)
    def _():
        acc_v[...] = jnp.zeros((G, H), _f32)

    acc_v[...] += lax.dot_general(oh, nf, (((0,), (0,)), ((), ())),
                                  preferred_element_type=_f32)
    hs0 = jnp.max(jnp.sum(nf * clws_r[0:1, :], axis=1))
    hs1 = jnp.max(jnp.sum(nf * clws_r[1:2, :], axis=1))

    @pl.when(i == 0)
    def _():
        mx_s[0] = hs0
        mx_s[1] = hs1

    @pl.when(i > 0)
    def _():
        mx_s[0] = jnp.maximum(mx_s[0], hs0)
        mx_s[1] = jnp.maximum(mx_s[1], hs1)

    @pl.when(i == NBN - 1)
    def _():
        g0_r[...] = acc_v[...]
        m0 = jnp.maximum(mx_s[0], hs0)
        m1 = jnp.maximum(mx_s[1], hs1)
        mhs_r[...] = jnp.broadcast_to(
            jnp.stack([m0, m1] + [jnp.float32(0.0)] * 6)[:, None], (8, H))


def _t6(tbl, den, hs, wihT, whhT, bih, bhh, n2g3, clws):
    return pl.pallas_call(
        _t6_body,
        grid=(NBN,),
        in_specs=[
            pl.BlockSpec((2, BN, H), lambda i: (0, i, 0)),
            pl.BlockSpec((2, BN, 1), lambda i: (0, i, 0)),
            pl.BlockSpec((BN, H), lambda i: (i, 0)),
            pl.BlockSpec((H, 384), lambda i: (0, 0)),
            pl.BlockSpec((H, 384), lambda i: (0, 0)),
            pl.BlockSpec((8, 384), lambda i: (0, 0)),
            pl.BlockSpec((8, 384), lambda i: (0, 0)),
            pl.BlockSpec((1, 1, BN), lambda i: (i, 0, 0)),
            pl.BlockSpec((8, H), lambda i: (0, 0)),
        ],
        out_specs=[
            pl.BlockSpec((BN, H), lambda i: (i, 0)),
            pl.BlockSpec((G, H), lambda i: (0, 0)),
            pl.BlockSpec((8, H), lambda i: (0, 0)),
        ],
        out_shape=[
            jax.ShapeDtypeStruct((N, H), _f32),
            jax.ShapeDtypeStruct((G, H), _f32),
            jax.ShapeDtypeStruct((8, H), _f32),
        ],
        scratch_shapes=[pltpu.SMEM((2,), _f32), pltpu.VMEM((G, H), _f32)],
    )(tbl, den, hs, wihT, whhT, bih, bhh, n2g3, clws)


def _t78_body(h_r, n2g_r, gf_r, scal_r, wpnT_r, bpn_r, wihT_r, whhT_r,
              bih_r, bhh_r, out_r, gsb_v, nacc_v, dacc_v):
    i = pl.program_id(0)

    @pl.when(i == 0)
    def _():
        gf = gf_r[...]
        gs = jnp.sum(jax.nn.relu(gf) * scal_r[0:1, :], axis=1,
                     keepdims=True) + scal_r[2:3, 0:1]
        gsb_v[...] = jnp.broadcast_to(gs, (G, H))
        nacc_v[...] = jnp.zeros((G, H), _f32)
        dacc_v[...] = jnp.zeros((G, H), _f32)

    hx = h_r[...]
    n2g = n2g_r[0, 0, :]
    oh = (n2g[:, None] == lax.broadcasted_iota(_i32, (BN, G), 1)).astype(_f32)
    gath = lax.dot_general(oh, gsb_v[...], (((1,), (0,)), ((), ())),
                           preferred_element_type=_f32)[:, 0:1]
    hscal = jnp.sum(hx * scal_r[1:2, :], axis=1, keepdims=True)
    z = _lrelu(gath + hscal)
    bd = _lrelu(gath + scal_r[3:4, 0:1])
    ex = jnp.exp(z - bd)
    hvp = _dot(hx, wpnT_r[...]) + bpn_r[0:1, :]
    nacc_v[...] += lax.dot_general(oh, ex * hvp, (((0,), (0,)), ((), ())),
                                   preferred_element_type=_f32)
    dacc_v[...] += lax.dot_general(oh, jnp.broadcast_to(ex, (BN, H)),
                                   (((0,), (0,)), ((), ())),
                                   preferred_element_type=_f32)

    @pl.when(i == NBN - 1)
    def _():
        grep = _elu(nacc_v[...] / (dacc_v[...][:, 0:1] + 1e-9))
        ctx = jax.nn.relu(grep)
        out_r[...] = _gru(ctx, gf_r[...], wihT_r[...], whhT_r[...],
                          bih_r[0:1, :], bhh_r[0:1, :])


def _t78(h_out, n2g3, gf, scal, wpnT, bpn, wihT, whhT, bih, bhh):
    return pl.pallas_call(
        _t78_body,
        grid=(NBN,),
        in_specs=[
            pl.BlockSpec((BN, H), lambda i: (i, 0)),
            pl.BlockSpec((1, 1, BN), lambda i: (i, 0, 0)),
            pl.BlockSpec((G, H), lambda i: (0, 0)),
            pl.BlockSpec((8, H), lambda i: (0, 0)),
            pl.BlockSpec((H, H), lambda i: (0, 0)),
            pl.BlockSpec((8, H), lambda i: (0, 0)),
            pl.BlockSpec((H, 384), lambda i: (0, 0)),
            pl.BlockSpec((H, 384), lambda i: (0, 0)),
            pl.BlockSpec((8, 384), lambda i: (0, 0)),
            pl.BlockSpec((8, 384), lambda i: (0, 0)),
        ],
        out_specs=pl.BlockSpec((G, H), lambda i: (0, 0)),
        out_shape=jax.ShapeDtypeStruct((G, H), _f32),
        scratch_shapes=[pltpu.VMEM((G, H), _f32), pltpu.VMEM((G, H), _f32),
                        pltpu.VMEM((G, H), _f32)],
    )(h_out, n2g3, gf, scal, wpnT, bpn, wihT, whhT, bih, bhh)


_SC_MESH = plsc.VectorSubcoreMesh(core_axis_name="c", subcore_axis_name="s")


@functools.partial(
    pl.kernel,
    out_type=jax.ShapeDtypeStruct((E, H), _f32),
    mesh=_SC_MESH,
    scratch_types=[
        pltpu.VMEM((CH,), _i32),
        pltpu.VMEM((CH, H), _f32),
        pltpu.VMEM((CH, H), _f32),
        pltpu.SemaphoreType.DMA,
    ],
)
def _sc_he1(hn_h, src_h, ee_h, out_h, idx_v, ra_v, rb_v, sem):
    core = lax.axis_index("c")
    sub = lax.axis_index("s")
    wid = core * 16 + sub

    def chunk(ci, carry):
        base = wid * EPW + ci * CH
        pltpu.sync_copy(src_h.at[pl.ds(base, CH)], idx_v)
        cp = pltpu.async_copy(hn_h.at[idx_v], ra_v, sem)
        pltpu.sync_copy(ee_h.at[pl.ds(base, CH)], rb_v)
        cp.wait()

        def row(r, c2):
            for k in range(8):
                a = ra_v[r, pl.ds(k * 16, 16)]
                b = rb_v[r, pl.ds(k * 16, 16)]
                x = a + b
                ra_v[r, pl.ds(k * 16, 16)] = jnp.where(x >= 0, x, 0.01 * x)
            return c2

        lax.fori_loop(0, CH, row, 0)
        pltpu.sync_copy(ra_v, out_h.at[pl.ds(base, CH)])
        return carry

    lax.fori_loop(0, NCH, chunk, 0)


def _sc_agg_body(rows_linear, dst_h, sd_h, mv_h, sev_src_h, rows_src_h,
                 src_h, out_h, outd_h,
                 didx_a, didx_b, sidx_a, sidx_b, sdv_a, sdv_b, sev_a, sev_b,
                 rows_a, rows_b, exv_v, scaled_v, mv_v, dbuf_v,
                 tbl_sh, den_sh, sem_a1, sem_a2, sem_a3, sem_b1, sem_b2,
                 sem_b3):
    core = lax.axis_index("c")
    sub = lax.axis_index("s")
    wid = core * 16 + sub

    def zrow(r, c):
        for k in range(H // 16):
            scaled_v[r, pl.ds(k * 16, 16)] = jnp.zeros((16,), _f32)
        return c

    lax.fori_loop(0, CH, zrow, 0)
    for i in range(RPS // CH):
        pltpu.sync_copy(scaled_v, tbl_sh.at[pl.ds(sub * RPS + i * CH, CH)])

    def zden(r, c):
        dbuf_v[pl.ds(r * 16, 16)] = jnp.zeros((16,), _f32)
        return c

    lax.fori_loop(0, RPS // 16, zden, 0)
    pltpu.sync_copy(dbuf_v, den_sh.at[pl.ds(sub * RPS, RPS)])
    plsc.subcore_barrier()

    pltpu.sync_copy(mv_h, mv_v)

    def issue(ci, didx_v, sidx_v, sdv_v, sev_v, rows_v, s1, s2, s3):
        base = wid * EPW + ci * CH
        pltpu.sync_copy(dst_h.at[pl.ds(base, CH)], didx_v)
        pltpu.async_copy(sd_h.at[didx_v], sdv_v, s1)
        if rows_linear:
            pltpu.async_copy(sev_src_h.at[pl.ds(base, CH)], sev_v, s2)
            pltpu.async_copy(rows_src_h.at[pl.ds(base, CH)], rows_v, s3)
        else:
            pltpu.sync_copy(src_h.at[pl.ds(base, CH)], sidx_v)
            pltpu.async_copy(sev_src_h.at[sidx_v], sev_v, s2)
            pltpu.async_copy(rows_src_h.at[sidx_v], rows_v, s3)

    def wait_set(ci, didx_v, sidx_v, sdv_v, sev_v, rows_v, s1, s2, s3):
        base = wid * EPW + ci * CH
        pltpu.make_async_copy(sd_h.at[didx_v], sdv_v, s1).wait()
        if rows_linear:
            pltpu.make_async_copy(
                sev_src_h.at[pl.ds(base, CH)], sev_v, s2).wait()
            pltpu.make_async_copy(
                rows_src_h.at[pl.ds(base, CH)], rows_v, s3).wait()
        else:
            pltpu.make_async_copy(sev_src_h.at[sidx_v], sev_v, s2).wait()
            pltpu.make_async_copy(rows_src_h.at[sidx_v], rows_v, s3).wait()

    def compute(didx_v, sdv_v, sev_v, rows_v):
        mv = mv_v[...]
        for g in range(CH // 16):
            s_d = sdv_v[pl.ds(g * 16, 16)]
            s_e = sev_v[pl.ds(g * 16, 16)]
            x = s_d + s_e
            lr = jnp.where(x >= 0, x, 0.01 * x)
            t2 = s_d + mv
            lb = jnp.where(t2 >= 0, t2, 0.01 * t2)
            exv_v[pl.ds(g * 16, 16)] = jnp.exp(lr - lb)
        for g in range(CH // 16):
            exg = exv_v[pl.ds(g * 16, 16)]
            for j in range(16):
                ei = g * 16 + j
                b = jnp.full((16,), exg[j], _f32)
                for k in range(H // 16):
                    scaled_v[ei, pl.ds(k * 16, 16)] = (
                        rows_v[ei, pl.ds(k * 16, 16)] * b)
        pltpu.sync_copy(scaled_v, tbl_sh.at[didx_v], add=True)
        pltpu.sync_copy(exv_v, den_sh.at[didx_v], add=True)

    seta = (didx_a, sidx_a, sdv_a, sev_a, rows_a, sem_a1, sem_a2, sem_a3)
    setb = (didx_b, sidx_b, sdv_b, sev_b, rows_b, sem_b1, sem_b2, sem_b3)

    issue(0, *seta)

    def chunk(ci, carry):
        @pl.when(ci % 2 == 0)
        def _():
            wait_set(ci, *seta)
            issue(ci + 1, *setb)
            compute(didx_a, sdv_a, sev_a, rows_a)

        @pl.when(ci % 2 == 1)
        def _():
            wait_set(ci, *setb)
            issue(ci + 1, *seta)
            compute(didx_b, sdv_b, sev_b, rows_b)

        return carry

    lax.fori_loop(0, NCH - 1, chunk, 0)
    wait_set(NCH - 1, *seta)
    compute(didx_a, sdv_a, sev_a, rows_a)

    plsc.subcore_barrier()
    for i in range(RPS // CH):
        pltpu.sync_copy(tbl_sh.at[pl.ds(sub * RPS + i * CH, CH)], scaled_v)
        pltpu.sync_copy(scaled_v,
                        out_h.at[core, pl.ds(sub * RPS + i * CH, CH)])
    pltpu.sync_copy(den_sh.at[pl.ds(sub * RPS, RPS)], dbuf_v)
    pltpu.sync_copy(dbuf_v, outd_h.at[core, pl.ds(sub * RPS, RPS)])


def _make_sc_agg(rows_linear):
    return functools.partial(
        pl.kernel,
        out_type=[jax.ShapeDtypeStruct((2, NPAD, H), _f32),
                  jax.ShapeDtypeStruct((2, NPAD), _f32)],
        mesh=_SC_MESH,
        scratch_types=[
            pltpu.VMEM((CH,), _i32),
            pltpu.VMEM((CH,), _i32),
            pltpu.VMEM((CH,), _i32),
            pltpu.VMEM((CH,), _i32),
            pltpu.VMEM((CH,), _f32),
            pltpu.VMEM((CH,), _f32),
            pltpu.VMEM((CH,), _f32),
            pltpu.VMEM((CH,), _f32),
            pltpu.VMEM((CH, H), _f32),
            pltpu.VMEM((CH, H), _f32),
            pltpu.VMEM((CH,), _f32),
            pltpu.VMEM((CH, H), _f32),
            pltpu.VMEM((16,), _f32),
            pltpu.VMEM((RPS,), _f32),
            pltpu.VMEM_SHARED((NPAD, H), _f32),
            pltpu.VMEM_SHARED((NPAD,), _f32),
            pltpu.SemaphoreType.DMA,
            pltpu.SemaphoreType.DMA,
            pltpu.SemaphoreType.DMA,
            pltpu.SemaphoreType.DMA,
            pltpu.SemaphoreType.DMA,
            pltpu.SemaphoreType.DMA,
        ],
    )(functools.partial(_sc_agg_body, rows_linear))


_sc_agg_linear = _make_sc_agg(True)
_sc_agg_gather = _make_sc_agg(False)


def kernel(h, e, edge_index, node2graph, params):
    p = params
    src = edge_index[0].astype(_i32)
    dst = edge_index[1].astype(_i32)
    n2g3 = node2graph.astype(_i32).reshape(NBN, 1, BN)

    h_pad = jnp.pad(h, ((0, 0), (0, 256 - h.shape[1])))
    wpn = jnp.pad(p['gc_pn_w'].T, ((0, 256 - h.shape[1]), (0, 0)))
    bpn = _pack8(p['gc_pn_b'], jnp.zeros((H,), _f32))
    wpe1n = jnp.pad(p['gc_pe1_w'][:, :133].T, ((0, 123), (0, 0)))
    scal1 = _pack8(p['gc_pe2_w'][0, :128],
                   jnp.full((H,), p['gc_pe2_b'][0], _f32))

    hv_new, hn, sd1b = _t1(h_pad, wpn, bpn, wpe1n, scal1)

    ee = _t2(e, p['gc_pe1_w'][:, 133:], _pack8(p['gc_pe1_b']))
    he1 = _sc_he1(hn, src, ee)
    m, se3, mse8 = _t3(he1, p['gc_et_w'].T, _pack8(p['gc_et_b']),
                       _pack8(p['gc_pe2_w'][0, 128:]))
    se = se3.reshape(E)
    sd1 = sd1b[:, 0]
    m16 = jnp.broadcast_to(mse8[0, 0], (16,))
    tbl, den = _sc_agg_linear(dst, sd1, m16, se, m, src)
    den = den.reshape(2, NPAD, 1)

    hs = hv_new
    wih = p['gc_gru_wih'].T
    whh = p['gc_gru_whh'].T
    bih = _pack8w(p['gc_gru_bih'])
    bhh = _pack8w(p['gc_gru_bhh'])
    for i in range(2):
        scal = _pack8(p['l_pe_w'][i][0, :128], p['l_pe_w'][i][0, 128:],
                      jnp.full((H,), p['l_pe_b'][i][0], _f32))
        nf, sdb, ssb, hvp, mssb = _t45(tbl, den, hs, wih, whh, bih, bhh, scal,
                                       p['l_pn_w'][i].T,
                                       _pack8(p['l_pn_b'][i]))
        m16 = jnp.broadcast_to(mssb[0, 0], (16,))
        tbl, den = _sc_agg_gather(dst, sdb[:, 0], m16, ssb[:, 0], hvp, src)
        den = den.reshape(2, NPAD, 1)
        hs = nf
        wih = p['l_gru_wih'][i].T
        whh = p['l_gru_whh'][i].T
        bih = _pack8w(p['l_gru_bih'][i])
        bhh = _pack8w(p['l_gru_bhh'][i])

    clws = _pack8(p['r_cl_w'][0][0, 128:], p['r_cl_w'][1][0, 128:])
    h_out, gf, mhs8 = _t6(tbl, den, hs, wih, whh, bih, bhh, n2g3, clws)

    for t in range(2):
        scal = _pack8(p['r_cl_w'][t][0, :128], p['r_cl_w'][t][0, 128:],
                      jnp.full((H,), p['r_cl_b'][t][0], _f32),
                      jnp.full((H,), mhs8[t, 0], _f32))
        gf = _t78(h_out, n2g3, gf, scal, p['r_pn_w'][t].T,
                  _pack8(p['r_pn_b'][t]),
                  p['r_gru_wih'][t].T, p['r_gru_whh'][t].T,
                  _pack8w(p['r_gru_bih'][t]), _pack8w(p['r_gru_bhh'][t]))
    return gf


# speedup vs baseline: 12.6747x; 1.1196x over previous
import functools

import jax
import jax.numpy as jnp
from jax import lax
from jax.experimental import pallas as pl
from jax.experimental.pallas import tpu as pltpu
from jax.experimental.pallas import tpu_sc as plsc

N = 10000
E = 320000
G = 256
H = 128
BN = 2000
NBN = N // BN
BE = 2000
NBE = E // BE
NW = 32
EPW = E // NW
CH = 80
NCH = EPW // CH
NPAD = 10240
RPS = NPAD // 16
ZCH = 128

_f32 = jnp.float32
_i32 = jnp.int32


def _lrelu(x):
    return jnp.where(x >= 0, x, 0.01 * x)


def _elu(x):
    return jnp.where(x > 0, x, jnp.exp(jnp.minimum(x, 0.0)) - 1.0)


def _pack8(*rows):
    rows = list(rows) + [jnp.zeros((128,), _f32)] * (8 - len(rows))
    return jnp.stack(rows)


def _pack8w(*rows):
    rows = list(rows) + [jnp.zeros((384,), _f32)] * (8 - len(rows))
    return jnp.stack(rows)


def _gru(x, hs, wihT, whhT, bih, bhh):
    gi = lax.dot_general(x, wihT, (((1,), (0,)), ((), ())),
                         preferred_element_type=_f32) + bih
    gh = lax.dot_general(hs, whhT, (((1,), (0,)), ((), ())),
                         preferred_element_type=_f32) + bhh
    r = jax.nn.sigmoid(gi[:, 0:128] + gh[:, 0:128])
    z = jax.nn.sigmoid(gi[:, 128:256] + gh[:, 128:256])
    n = jnp.tanh(gi[:, 256:384] + r * gh[:, 256:384])
    return (1.0 - z) * n + z * hs


def _dot(a, b):
    return lax.dot_general(a, b, (((1,), (0,)), ((), ())),
                           preferred_element_type=_f32)


def _t1_body(h_r, wpn_r, bpn_r, wpe1n_r, scal_r, hv_r, hn_r, sd_r):
    x = h_r[...]
    hv = _lrelu(_dot(x, wpn_r[...]) + bpn_r[0:1, :])
    hv_r[...] = hv
    hn_r[...] = _dot(x, wpe1n_r[...])
    sd = jnp.sum(hv * scal_r[0:1, :], axis=1, keepdims=True) + scal_r[1:2, 0:1]
    sd_r[...] = jnp.broadcast_to(sd, (BN, H))


def _t1(h_pad, wpn, bpn, wpe1n, scal):
    return pl.pallas_call(
        _t1_body,
        grid=(NBN,),
        in_specs=[
            pl.BlockSpec((BN, 256), lambda i: (i, 0)),
            pl.BlockSpec((256, H), lambda i: (0, 0)),
            pl.BlockSpec((8, H), lambda i: (0, 0)),
            pl.BlockSpec((256, H), lambda i: (0, 0)),
            pl.BlockSpec((8, H), lambda i: (0, 0)),
        ],
        out_specs=[
            pl.BlockSpec((BN, H), lambda i: (i, 0)),
            pl.BlockSpec((BN, H), lambda i: (i, 0)),
            pl.BlockSpec((BN, H), lambda i: (i, 0)),
        ],
        out_shape=[jax.ShapeDtypeStruct((N, H), _f32)] * 3,
    )(h_pad, wpn, bpn, wpe1n, scal)


def _t2_body(e_r, we_r, be_r, ee_r):
    ee_r[...] = lax.dot_general(e_r[...], we_r[...], (((1,), (1,)), ((), ())),
                                preferred_element_type=_f32) + be_r[0:1, :]


def _t2(e, we, be):
    fe = e.shape[1]
    return pl.pallas_call(
        _t2_body,
        grid=(NBE,),
        in_specs=[
            pl.BlockSpec((BE, fe), lambda i: (i, 0)),
            pl.BlockSpec((H, fe), lambda i: (0, 0)),
            pl.BlockSpec((8, H), lambda i: (0, 0)),
        ],
        out_specs=pl.BlockSpec((BE, H), lambda i: (i, 0)),
        out_shape=jax.ShapeDtypeStruct((E, H), _f32),
    )(e, we, be)


def _t3_body(he_r, wet_r, bet_r, w2b_r, m_r, se_r, mse_r, mx_s):
    i = pl.program_id(0)
    h1 = he_r[...]
    m_r[...] = _dot(h1, wet_r[...]) + bet_r[0:1, :]
    se_row = lax.dot_general(w2b_r[0:1, :], h1, (((1,), (1,)), ((), ())),
                             preferred_element_type=_f32)
    se_r[...] = se_row.reshape(1, 1, BE)
    bmax = jnp.max(se_row)

    @pl.when(i == 0)
    def _():
        mx_s[0] = bmax

    @pl.when(i > 0)
    def _():
        mx_s[0] = jnp.maximum(mx_s[0], bmax)

    @pl.when(i == NBE - 1)
    def _():
        mse_r[...] = jnp.broadcast_to(jnp.maximum(mx_s[0], bmax), (8, H))


def _t3(he1, wet, bet, w2b):
    return pl.pallas_call(
        _t3_body,
        grid=(NBE,),
        in_specs=[
            pl.BlockSpec((BE, H), lambda i: (i, 0)),
            pl.BlockSpec((H, H), lambda i: (0, 0)),
            pl.BlockSpec((8, H), lambda i: (0, 0)),
            pl.BlockSpec((8, H), lambda i: (0, 0)),
        ],
        out_specs=[
            pl.BlockSpec((BE, H), lambda i: (i, 0)),
            pl.BlockSpec((1, 1, BE), lambda i: (i, 0, 0)),
            pl.BlockSpec((8, H), lambda i: (0, 0)),
        ],
        out_shape=[
            jax.ShapeDtypeStruct((E, H), _f32),
            jax.ShapeDtypeStruct((NBE, 1, BE), _f32),
            jax.ShapeDtypeStruct((8, H), _f32),
        ],
        scratch_shapes=[pltpu.SMEM((1,), _f32)],
    )(he1, wet, bet, w2b)


def _t45_body(tbl_r, den_r, hs_r, wihT_r, whhT_r, bih_r, bhh_r, scal_r,
              wpnT_r, bpn_r, nf_r, sd_r, ss_r, hvp_r, mss_r, mx_s):
    i = pl.program_id(0)
    t = tbl_r[...]
    numer = t[0] + t[1]
    d = den_r[...]
    den = d[0] + d[1]
    ctx = _elu(numer / (den + 1e-9))
    nf = jax.nn.relu(_gru(ctx, hs_r[...], wihT_r[...], whhT_r[...],
                          bih_r[0:1, :], bhh_r[0:1, :]))
    nf_r[...] = nf
    sd = jnp.sum(nf * scal_r[0:1, :], axis=1, keepdims=True) + scal_r[2:3, 0:1]
    sd_r[...] = jnp.broadcast_to(sd, (BN, H))
    ss = jnp.sum(nf * scal_r[1:2, :], axis=1, keepdims=True)
    ss_r[...] = jnp.broadcast_to(ss, (BN, H))
    hvp_r[...] = _dot(nf, wpnT_r[...]) + bpn_r[0:1, :]
    bmax = jnp.max(ss)

    @pl.when(i == 0)
    def _():
        mx_s[0] = bmax

    @pl.when(i > 0)
    def _():
        mx_s[0] = jnp.maximum(mx_s[0], bmax)

    @pl.when(i == NBN - 1)
    def _():
        mss_r[...] = jnp.broadcast_to(jnp.maximum(mx_s[0], bmax), (8, H))


def _t45(tbl, den, hs, wihT, whhT, bih, bhh, scal, wpnT, bpn):
    return pl.pallas_call(
        _t45_body,
        grid=(NBN,),
        in_specs=[
            pl.BlockSpec((2, BN, H), lambda i: (0, i, 0)),
            pl.BlockSpec((2, BN, 1), lambda i: (0, i, 0)),
            pl.BlockSpec((BN, H), lambda i: (i, 0)),
            pl.BlockSpec((H, 384), lambda i: (0, 0)),
            pl.BlockSpec((H, 384), lambda i: (0, 0)),
            pl.BlockSpec((8, 384), lambda i: (0, 0)),
            pl.BlockSpec((8, 384), lambda i: (0, 0)),
            pl.BlockSpec((8, H), lambda i: (0, 0)),
            pl.BlockSpec((H, H), lambda i: (0, 0)),
            pl.BlockSpec((8, H), lambda i: (0, 0)),
        ],
        out_specs=[
            pl.BlockSpec((BN, H), lambda i: (i, 0)),
            pl.BlockSpec((BN, H), lambda i: (i, 0)),
            pl.BlockSpec((BN, H), lambda i: (i, 0)),
            pl.BlockSpec((BN, H), lambda i: (i, 0)),
            pl.BlockSpec((8, H), lambda i: (0, 0)),
        ],
        out_shape=[
            jax.ShapeDtypeStruct((N, H), _f32),
            jax.ShapeDtypeStruct((N, H), _f32),
            jax.ShapeDtypeStruct((N, H), _f32),
            jax.ShapeDtypeStruct((N, H), _f32),
            jax.ShapeDtypeStruct((8, H), _f32),
        ],
        scratch_shapes=[pltpu.SMEM((1,), _f32)],
    )(tbl, den, hs, wihT, whhT, bih, bhh, scal, wpnT, bpn)


def _t6_body(tbl_r, den_r, hs_r, wihT_r, whhT_r, bih_r, bhh_r, n2g_r, clws_r,
             h_r, g0_r, mhs_r, mx_s, acc_v):
    i = pl.program_id(0)
    t = tbl_r[...]
    numer = t[0] + t[1]
    d = den_r[...]
    den = d[0] + d[1]
    ctx = _elu(numer / (den + 1e-9))
    nf = jax.nn.relu(_gru(ctx, hs_r[...], wihT_r[...], whhT_r[...],
                          bih_r[0:1, :], bhh_r[0:1, :]))
    h_r[...] = nf
    n2g = n2g_r[0, 0, :]
    oh = (n2g[:, None] == lax.broadcasted_iota(_i32, (BN, G), 1)).astype(_f32)

    @pl.when(i == 0)
    def _():
        acc_v[...] = jnp.zeros((G, H), _f32)

    acc_v[...] += lax.dot_general(oh, nf, (((0,), (0,)), ((), ())),
                                  preferred_element_type=_f32)
    hs0 = jnp.max(jnp.sum(nf * clws_r[0:1, :], axis=1))
    hs1 = jnp.max(jnp.sum(nf * clws_r[1:2, :], axis=1))

    @pl.when(i == 0)
    def _():
        mx_s[0] = hs0
        mx_s[1] = hs1

    @pl.when(i > 0)
    def _():
        mx_s[0] = jnp.maximum(mx_s[0], hs0)
        mx_s[1] = jnp.maximum(mx_s[1], hs1)

    @pl.when(i == NBN - 1)
    def _():
        g0_r[...] = acc_v[...]
        m0 = jnp.maximum(mx_s[0], hs0)
        m1 = jnp.maximum(mx_s[1], hs1)
        mhs_r[...] = jnp.broadcast_to(
            jnp.stack([m0, m1] + [jnp.float32(0.0)] * 6)[:, None], (8, H))


def _t6(tbl, den, hs, wihT, whhT, bih, bhh, n2g3, clws):
    return pl.pallas_call(
        _t6_body,
        grid=(NBN,),
        in_specs=[
            pl.BlockSpec((2, BN, H), lambda i: (0, i, 0)),
            pl.BlockSpec((2, BN, 1), lambda i: (0, i, 0)),
            pl.BlockSpec((BN, H), lambda i: (i, 0)),
            pl.BlockSpec((H, 384), lambda i: (0, 0)),
            pl.BlockSpec((H, 384), lambda i: (0, 0)),
            pl.BlockSpec((8, 384), lambda i: (0, 0)),
            pl.BlockSpec((8, 384), lambda i: (0, 0)),
            pl.BlockSpec((1, 1, BN), lambda i: (i, 0, 0)),
            pl.BlockSpec((8, H), lambda i: (0, 0)),
        ],
        out_specs=[
            pl.BlockSpec((BN, H), lambda i: (i, 0)),
            pl.BlockSpec((G, H), lambda i: (0, 0)),
            pl.BlockSpec((8, H), lambda i: (0, 0)),
        ],
        out_shape=[
            jax.ShapeDtypeStruct((N, H), _f32),
            jax.ShapeDtypeStruct((G, H), _f32),
            jax.ShapeDtypeStruct((8, H), _f32),
        ],
        scratch_shapes=[pltpu.SMEM((2,), _f32), pltpu.VMEM((G, H), _f32)],
    )(tbl, den, hs, wihT, whhT, bih, bhh, n2g3, clws)


def _t78_body(h_r, n2g_r, gf_r, scal_r, wpnT_r, bpn_r, wihT_r, whhT_r,
              bih_r, bhh_r, out_r, gsb_v, nacc_v, dacc_v):
    i = pl.program_id(0)

    @pl.when(i == 0)
    def _():
        gf = gf_r[...]
        gs = jnp.sum(jax.nn.relu(gf) * scal_r[0:1, :], axis=1,
                     keepdims=True) + scal_r[2:3, 0:1]
        gsb_v[...] = jnp.broadcast_to(gs, (G, H))
        nacc_v[...] = jnp.zeros((G, H), _f32)
        dacc_v[...] = jnp.zeros((G, H), _f32)

    hx = h_r[...]
    n2g = n2g_r[0, 0, :]
    oh = (n2g[:, None] == lax.broadcasted_iota(_i32, (BN, G), 1)).astype(_f32)
    gath = lax.dot_general(oh, gsb_v[...], (((1,), (0,)), ((), ())),
                           preferred_element_type=_f32)[:, 0:1]
    hscal = jnp.sum(hx * scal_r[1:2, :], axis=1, keepdims=True)
    z = _lrelu(gath + hscal)
    bd = _lrelu(gath + scal_r[3:4, 0:1])
    ex = jnp.exp(z - bd)
    hvp = _dot(hx, wpnT_r[...]) + bpn_r[0:1, :]
    nacc_v[...] += lax.dot_general(oh, ex * hvp, (((0,), (0,)), ((), ())),
                                   preferred_element_type=_f32)
    dacc_v[...] += lax.dot_general(oh, jnp.broadcast_to(ex, (BN, H)),
                                   (((0,), (0,)), ((), ())),
                                   preferred_element_type=_f32)

    @pl.when(i == NBN - 1)
    def _():
        grep = _elu(nacc_v[...] / (dacc_v[...][:, 0:1] + 1e-9))
        ctx = jax.nn.relu(grep)
        out_r[...] = _gru(ctx, gf_r[...], wihT_r[...], whhT_r[...],
                          bih_r[0:1, :], bhh_r[0:1, :])


def _t78(h_out, n2g3, gf, scal, wpnT, bpn, wihT, whhT, bih, bhh):
    return pl.pallas_call(
        _t78_body,
        grid=(NBN,),
        in_specs=[
            pl.BlockSpec((BN, H), lambda i: (i, 0)),
            pl.BlockSpec((1, 1, BN), lambda i: (i, 0, 0)),
            pl.BlockSpec((G, H), lambda i: (0, 0)),
            pl.BlockSpec((8, H), lambda i: (0, 0)),
            pl.BlockSpec((H, H), lambda i: (0, 0)),
            pl.BlockSpec((8, H), lambda i: (0, 0)),
            pl.BlockSpec((H, 384), lambda i: (0, 0)),
            pl.BlockSpec((H, 384), lambda i: (0, 0)),
            pl.BlockSpec((8, 384), lambda i: (0, 0)),
            pl.BlockSpec((8, 384), lambda i: (0, 0)),
        ],
        out_specs=pl.BlockSpec((G, H), lambda i: (0, 0)),
        out_shape=jax.ShapeDtypeStruct((G, H), _f32),
        scratch_shapes=[pltpu.VMEM((G, H), _f32), pltpu.VMEM((G, H), _f32),
                        pltpu.VMEM((G, H), _f32)],
    )(h_out, n2g3, gf, scal, wpnT, bpn, wihT, whhT, bih, bhh)


_SC_MESH = plsc.VectorSubcoreMesh(core_axis_name="c", subcore_axis_name="s")


@functools.partial(
    pl.kernel,
    out_type=jax.ShapeDtypeStruct((E, H), _f32),
    mesh=_SC_MESH,
    scratch_types=[
        pltpu.VMEM((CH,), _i32),
        pltpu.VMEM((CH,), _i32),
        pltpu.VMEM((CH, H), _f32),
        pltpu.VMEM((CH, H), _f32),
        pltpu.VMEM((CH, H), _f32),
        pltpu.VMEM((CH, H), _f32),
        pltpu.SemaphoreType.DMA,
        pltpu.SemaphoreType.DMA,
        pltpu.SemaphoreType.DMA,
        pltpu.SemaphoreType.DMA,
        pltpu.SemaphoreType.DMA,
        pltpu.SemaphoreType.DMA,
    ],
)
def _sc_he1(hn_h, src_h, ee_h, out_h, idx_a, idx_b, ra_a, ra_b, rb_a, rb_b,
            s_a1, s_a2, s_ao, s_b1, s_b2, s_bo):
    core = lax.axis_index("c")
    sub = lax.axis_index("s")
    wid = core * 16 + sub

    def issue(ci, idx_v, ra_v, rb_v, s1, s2, so):
        base = wid * EPW + ci * CH
        pltpu.sync_copy(src_h.at[pl.ds(base, CH)], idx_v)
        pltpu.async_copy(hn_h.at[idx_v], ra_v, s1)
        pltpu.async_copy(ee_h.at[pl.ds(base, CH)], rb_v, s2)

    def compute(ci, idx_v, ra_v, rb_v, s1, s2, so):
        base = wid * EPW + ci * CH
        pltpu.make_async_copy(hn_h.at[idx_v], ra_v, s1).wait()
        pltpu.make_async_copy(ee_h.at[pl.ds(base, CH)], rb_v, s2).wait()

        def row(r, c2):
            for k in range(8):
                a = ra_v[r, pl.ds(k * 16, 16)]
                b = rb_v[r, pl.ds(k * 16, 16)]
                x = a + b
                ra_v[r, pl.ds(k * 16, 16)] = jnp.where(x >= 0, x, 0.01 * x)
            return c2

        lax.fori_loop(0, CH, row, 0)
        pltpu.async_copy(ra_v, out_h.at[pl.ds(base, CH)], so)

    def wait_out(ci, ra_v, so):
        base = wid * EPW + ci * CH
        pltpu.make_async_copy(ra_v, out_h.at[pl.ds(base, CH)], so).wait()

    seta = (idx_a, ra_a, rb_a, s_a1, s_a2, s_ao)
    setb = (idx_b, ra_b, rb_b, s_b1, s_b2, s_bo)

    issue(0, *seta)

    def chunk(ci, carry):
        @pl.when(ci % 2 == 0)
        def _():
            issue(ci + 1, *setb)

            @pl.when(ci >= 2)
            def _():
                wait_out(ci - 2, ra_a, s_ao)

            compute(ci, *seta)

        @pl.when(ci % 2 == 1)
        def _():
            issue(ci + 1, *seta)

            @pl.when(ci >= 2)
            def _():
                wait_out(ci - 2, ra_b, s_bo)

            compute(ci, *setb)

        return carry

    lax.fori_loop(0, NCH - 1, chunk, 0)
    wait_out(NCH - 3, ra_a, s_ao)
    compute(NCH - 1, *seta)
    wait_out(NCH - 2, ra_b, s_bo)
    wait_out(NCH - 1, ra_a, s_ao)


def _sc_agg_body(rows_linear, dst_h, sd_h, mv_h, sev_src_h, rows_src_h,
                 src_h, out_h, outd_h,
                 didx_a, didx_b, sidx_a, sidx_b, sdv_a, sdv_b, sev_a, sev_b,
                 rows_a, rows_b, exv_v, scaled_v, mv_v, dbuf_v,
                 tbl_sh, den_sh, sem_a1, sem_a2, sem_a3, sem_b1, sem_b2,
                 sem_b3):
    core = lax.axis_index("c")
    sub = lax.axis_index("s")
    wid = core * 16 + sub

    def zrow(r, c):
        for k in range(H // 16):
            scaled_v[r, pl.ds(k * 16, 16)] = jnp.zeros((16,), _f32)
        return c

    lax.fori_loop(0, CH, zrow, 0)
    for i in range(RPS // CH):
        pltpu.sync_copy(scaled_v, tbl_sh.at[pl.ds(sub * RPS + i * CH, CH)])

    def zden(r, c):
        dbuf_v[pl.ds(r * 16, 16)] = jnp.zeros((16,), _f32)
        return c

    lax.fori_loop(0, RPS // 16, zden, 0)
    pltpu.sync_copy(dbuf_v, den_sh.at[pl.ds(sub * RPS, RPS)])
    plsc.subcore_barrier()

    pltpu.sync_copy(mv_h, mv_v)

    def issue(ci, didx_v, sidx_v, sdv_v, sev_v, rows_v, s1, s2, s3):
        base = wid * EPW + ci * CH
        pltpu.sync_copy(dst_h.at[pl.ds(base, CH)], didx_v)
        pltpu.async_copy(sd_h.at[didx_v], sdv_v, s1)
        if rows_linear:
            pltpu.async_copy(sev_src_h.at[pl.ds(base, CH)], sev_v, s2)
            pltpu.async_copy(rows_src_h.at[pl.ds(base, CH)], rows_v, s3)
        else:
            pltpu.sync_copy(src_h.at[pl.ds(base, CH)], sidx_v)
            pltpu.async_copy(sev_src_h.at[sidx_v], sev_v, s2)
            pltpu.async_copy(rows_src_h.at[sidx_v], rows_v, s3)

    def wait_set(ci, didx_v, sidx_v, sdv_v, sev_v, rows_v, s1, s2, s3):
        base = wid * EPW + ci * CH
        pltpu.make_async_copy(sd_h.at[didx_v], sdv_v, s1).wait()
        if rows_linear:
            pltpu.make_async_copy(
                sev_src_h.at[pl.ds(base, CH)], sev_v, s2).wait()
            pltpu.make_async_copy(
                rows_src_h.at[pl.ds(base, CH)], rows_v, s3).wait()
        else:
            pltpu.make_async_copy(sev_src_h.at[sidx_v], sev_v, s2).wait()
            pltpu.make_async_copy(rows_src_h.at[sidx_v], rows_v, s3).wait()

    def compute(didx_v, sdv_v, sev_v, rows_v):
        mv = mv_v[...]
        for g in range(CH // 16):
            s_d = sdv_v[pl.ds(g * 16, 16)]
            s_e = sev_v[pl.ds(g * 16, 16)]
            x = s_d + s_e
            lr = jnp.where(x >= 0, x, 0.01 * x)
            t2 = s_d + mv
            lb = jnp.where(t2 >= 0, t2, 0.01 * t2)
            exv_v[pl.ds(g * 16, 16)] = jnp.exp(lr - lb)
        for g in range(CH // 16):
            exg = exv_v[pl.ds(g * 16, 16)]
            for j in range(16):
                ei = g * 16 + j
                b = jnp.full((16,), exg[j], _f32)
                for k in range(H // 16):
                    scaled_v[ei, pl.ds(k * 16, 16)] = (
                        rows_v[ei, pl.ds(k * 16, 16)] * b)
        pltpu.sync_copy(scaled_v, tbl_sh.at[didx_v], add=True)
        pltpu.sync_copy(exv_v, den_sh.at[didx_v], add=True)

    seta = (didx_a, sidx_a, sdv_a, sev_a, rows_a, sem_a1, sem_a2, sem_a3)
    setb = (didx_b, sidx_b, sdv_b, sev_b, rows_b, sem_b1, sem_b2, sem_b3)

    issue(0, *seta)

    def chunk(ci, carry):
        @pl.when(ci % 2 == 0)
        def _():
            wait_set(ci, *seta)
            issue(ci + 1, *setb)
            compute(didx_a, sdv_a, sev_a, rows_a)

        @pl.when(ci % 2 == 1)
        def _():
            wait_set(ci, *setb)
            issue(ci + 1, *seta)
            compute(didx_b, sdv_b, sev_b, rows_b)

        return carry

    lax.fori_loop(0, NCH - 1, chunk, 0)
    wait_set(NCH - 1, *seta)
    compute(didx_a, sdv_a, sev_a, rows_a)

    plsc.subcore_barrier()
    for i in range(RPS // CH):
        pltpu.sync_copy(tbl_sh.at[pl.ds(sub * RPS + i * CH, CH)], scaled_v)
        pltpu.sync_copy(scaled_v,
                        out_h.at[core, pl.ds(sub * RPS + i * CH, CH)])
    pltpu.sync_copy(den_sh.at[pl.ds(sub * RPS, RPS)], dbuf_v)
    pltpu.sync_copy(dbuf_v, outd_h.at[core, pl.ds(sub * RPS, RPS)])


def _make_sc_agg(rows_linear):
    return functools.partial(
        pl.kernel,
        out_type=[jax.ShapeDtypeStruct((2, NPAD, H), _f32),
                  jax.ShapeDtypeStruct((2, NPAD), _f32)],
        mesh=_SC_MESH,
        scratch_types=[
            pltpu.VMEM((CH,), _i32),
            pltpu.VMEM((CH,), _i32),
            pltpu.VMEM((CH,), _i32),
            pltpu.VMEM((CH,), _i32),
            pltpu.VMEM((CH,), _f32),
            pltpu.VMEM((CH,), _f32),
            pltpu.VMEM((CH,), _f32),
            pltpu.VMEM((CH,), _f32),
            pltpu.VMEM((CH, H), _f32),
            pltpu.VMEM((CH, H), _f32),
            pltpu.VMEM((CH,), _f32),
            pltpu.VMEM((CH, H), _f32),
            pltpu.VMEM((16,), _f32),
            pltpu.VMEM((RPS,), _f32),
            pltpu.VMEM_SHARED((NPAD, H), _f32),
            pltpu.VMEM_SHARED((NPAD,), _f32),
            pltpu.SemaphoreType.DMA,
            pltpu.SemaphoreType.DMA,
            pltpu.SemaphoreType.DMA,
            pltpu.SemaphoreType.DMA,
            pltpu.SemaphoreType.DMA,
            pltpu.SemaphoreType.DMA,
        ],
    )(functools.partial(_sc_agg_body, rows_linear))


_sc_agg_linear = _make_sc_agg(True)
_sc_agg_gather = _make_sc_agg(False)


def kernel(h, e, edge_index, node2graph, params):
    p = params
    src = edge_index[0].astype(_i32)
    dst = edge_index[1].astype(_i32)
    n2g3 = node2graph.astype(_i32).reshape(NBN, 1, BN)

    h_pad = jnp.pad(h, ((0, 0), (0, 256 - h.shape[1])))
    wpn = jnp.pad(p['gc_pn_w'].T, ((0, 256 - h.shape[1]), (0, 0)))
    bpn = _pack8(p['gc_pn_b'], jnp.zeros((H,), _f32))
    wpe1n = jnp.pad(p['gc_pe1_w'][:, :133].T, ((0, 123), (0, 0)))
    scal1 = _pack8(p['gc_pe2_w'][0, :128],
                   jnp.full((H,), p['gc_pe2_b'][0], _f32))

    hv_new, hn, sd1b = _t1(h_pad, wpn, bpn, wpe1n, scal1)

    ee = _t2(e, p['gc_pe1_w'][:, 133:], _pack8(p['gc_pe1_b']))
    he1 = _sc_he1(hn, src, ee)
    m, se3, mse8 = _t3(he1, p['gc_et_w'].T, _pack8(p['gc_et_b']),
                       _pack8(p['gc_pe2_w'][0, 128:]))
    se = se3.reshape(E)
    sd1 = sd1b[:, 0]
    m16 = jnp.broadcast_to(mse8[0, 0], (16,))
    tbl, den = _sc_agg_linear(dst, sd1, m16, se, m, src)
    den = den.reshape(2, NPAD, 1)

    hs = hv_new
    wih = p['gc_gru_wih'].T
    whh = p['gc_gru_whh'].T
    bih = _pack8w(p['gc_gru_bih'])
    bhh = _pack8w(p['gc_gru_bhh'])
    for i in range(2):
        scal = _pack8(p['l_pe_w'][i][0, :128], p['l_pe_w'][i][0, 128:],
                      jnp.full((H,), p['l_pe_b'][i][0], _f32))
        nf, sdb, ssb, hvp, mssb = _t45(tbl, den, hs, wih, whh, bih, bhh, scal,
                                       p['l_pn_w'][i].T,
                                       _pack8(p['l_pn_b'][i]))
        m16 = jnp.broadcast_to(mssb[0, 0], (16,))
        tbl, den = _sc_agg_gather(dst, sdb[:, 0], m16, ssb[:, 0], hvp, src)
        den = den.reshape(2, NPAD, 1)
        hs = nf
        wih = p['l_gru_wih'][i].T
        whh = p['l_gru_whh'][i].T
        bih = _pack8w(p['l_gru_bih'][i])
        bhh = _pack8w(p['l_gru_bhh'][i])

    clws = _pack8(p['r_cl_w'][0][0, 128:], p['r_cl_w'][1][0, 128:])
    h_out, gf, mhs8 = _t6(tbl, den, hs, wih, whh, bih, bhh, n2g3, clws)

    for t in range(2):
        scal = _pack8(p['r_cl_w'][t][0, :128], p['r_cl_w'][t][0, 128:],
                      jnp.full((H,), p['r_cl_b'][t][0], _f32),
                      jnp.full((H,), mhs8[t, 0], _f32))
        gf = _t78(h_out, n2g3, gf, scal, p['r_pn_w'][t].T,
                  _pack8(p['r_pn_b'][t]),
                  p['r_gru_wih'][t].T, p['r_gru_whh'][t].T,
                  _pack8w(p['r_gru_bih'][t]), _pack8w(p['r_gru_bhh'][t]))
    return gf
